# Initial kernel scaffold; baseline (speedup 1.0000x reference)
import jax
import jax.numpy as jnp
from jax.experimental import pallas as pl


def kernel(x, edge_index, src, dst, W_rwse, b_rwse, W_in, b_in, Wq0, bq0, Wk0, bk0, Wv0, bv0, Ws0, bs0, g0, be0, Wq1, bq1, Wk1, bk1, Wv1, bv1, Ws1, bs1, g1, be1):
    raise NotImplementedError("write your pallas kernel here")


# speedup vs baseline: 9.3192x; 9.3192x over previous
import functools

import jax
import jax.numpy as jnp
from jax import lax
from jax.experimental import pallas as pl
from jax.experimental.pallas import tpu as pltpu
from jax.experimental.pallas import tpu_sc as plsc

N = 2048
E = 32768
D_FEAT = 128
HID = 64
HEADS = 4
DH = 16
WALK = 16
RWSE_DIM = 16
NPAIRS = 4096

_SC_PARAMS = pltpu.CompilerParams(needs_layout_passes=False)


@functools.cache
def _sc_mesh():
    return plsc.VectorSubcoreMesh(core_axis_name="c", subcore_axis_name="s")


ROWS_PER_TILE = 64
COLS_PER_PASS = 1024
ECHUNK = 8192


@functools.cache
def _build_counts_kernel():
    return functools.partial(
        pl.kernel,
        out_type=jax.ShapeDtypeStruct((N, N), jnp.float32),
        mesh=_sc_mesh(),
        scratch_types=[
            pltpu.VMEM((ROWS_PER_TILE, COLS_PER_PASS), jnp.float32),
            pltpu.VMEM((ECHUNK,), jnp.int32),
            pltpu.VMEM((ECHUNK,), jnp.int32),
        ],
        compiler_params=_SC_PARAMS,
    )(_build_counts_body)


def _build_counts(row, col):
    return _build_counts_kernel()(row, col)


def _build_counts_body(row_hbm, col_hbm, out_hbm, acc_v, r_v, c_v):
    wid = lax.axis_index("s") * 2 + lax.axis_index("c")
    r0 = wid * ROWS_PER_TILE
    zeros16 = jnp.zeros((16,), jnp.float32)
    ones16 = jnp.ones((16,), jnp.float32)
    for p in range(N // COLS_PER_PASS):
        c0 = p * COLS_PER_PASS

        def zero_body(i, _):
            r = i // (COLS_PER_PASS // 16)
            c = (i % (COLS_PER_PASS // 16)) * 16
            acc_v[r, pl.ds(c, 16)] = zeros16
            return 0

        lax.fori_loop(0, ROWS_PER_TILE * COLS_PER_PASS // 16, zero_body, 0)

        def chunk_body(ch, _):
            pltpu.sync_copy(row_hbm.at[pl.ds(ch * ECHUNK, ECHUNK)], r_v)
            pltpu.sync_copy(col_hbm.at[pl.ds(ch * ECHUNK, ECHUNK)], c_v)

            def group_body(g, _):
                r16 = r_v[pl.ds(g * 16, 16)]
                c16 = c_v[pl.ds(g * 16, 16)]
                m = ((r16 >= r0) & (r16 < r0 + ROWS_PER_TILE)
                     & (c16 >= c0) & (c16 < c0 + COLS_PER_PASS))
                rr = jnp.where(m, r16 - r0, 0)
                cc = jnp.where(m, c16 - c0, 0)
                plsc.addupdate_scatter(acc_v, [rr, cc], ones16, mask=m)
                return 0

            lax.fori_loop(0, ECHUNK // 16, group_body, 0)
            return 0

        lax.fori_loop(0, E // ECHUNK, chunk_body, 0)
        pltpu.sync_copy(
            acc_v, out_hbm.at[pl.ds(r0, ROWS_PER_TILE), pl.ds(c0, COLS_PER_PASS)])


def _norm_kernel(a_ref, p_ref):
    a = a_ref[...]
    deg = jnp.sum(a, axis=1, keepdims=True)
    p_ref[...] = a / jnp.maximum(deg, 1.0)


def _normalize(a):
    return pl.pallas_call(
        _norm_kernel,
        grid=(8,),
        in_specs=[pl.BlockSpec((N // 8, N), lambda i: (i, 0))],
        out_specs=pl.BlockSpec((N // 8, N), lambda i: (i, 0)),
        out_shape=jax.ShapeDtypeStruct((N, N), jnp.float32),
    )(a)


_MB = 512


def _mm_kernel(a_ref, b_ref, o_ref):
    @pl.when(pl.program_id(2) == 0)
    def _():
        o_ref[...] = jnp.zeros_like(o_ref)

    o_ref[...] += jnp.dot(a_ref[...], b_ref[...],
                          preferred_element_type=jnp.float32)


def _mm(a, b):
    g = N // _MB
    return pl.pallas_call(
        _mm_kernel,
        grid=(g, g, g),
        in_specs=[pl.BlockSpec((_MB, _MB), lambda i, j, k: (i, k)),
                  pl.BlockSpec((_MB, _MB), lambda i, j, k: (k, j))],
        out_specs=pl.BlockSpec((_MB, _MB), lambda i, j, k: (i, j)),
        out_shape=jax.ShapeDtypeStruct((N, N), jnp.float32),
    )(a, b)


_BT = 128


def _diag_kernel(p1_ref, p2_ref, p3_ref, p4a_ref, p4b_ref, p8_ref, p12_ref,
                 o_ref):
    bi = pl.program_id(0)
    bj = pl.program_id(1)

    @pl.when(bj == 0)
    def _():
        o_ref[...] = jnp.zeros_like(o_ref)

    ii = lax.broadcasted_iota(jnp.int32, (_BT, _BT), 0)
    jj = lax.broadcasted_iota(jnp.int32, (_BT, _BT), 1)
    eye = (ii == jj).astype(jnp.float32)

    def tr(x):
        return lax.dot_general(x, eye, (((0,), (0,)), ((), ())),
                               preferred_element_type=jnp.float32)

    t4 = tr(p4b_ref[...])
    t8 = tr(p8_ref[...])
    t12 = tr(p12_ref[...])
    a1 = p1_ref[...]
    a2 = p2_ref[...]
    a3 = p3_ref[...]
    a4 = p4a_ref[...]
    flag = jnp.where(bi == bj, 1.0, 0.0)

    def rs(x):
        return jnp.sum(x, axis=1, keepdims=True)

    cols = [
        rs(a1 * eye) * flag,
        rs(a2 * eye) * flag,
        rs(a3 * eye) * flag,
        rs(a4 * eye) * flag,
        rs(a1 * t4),
        rs(a2 * t4),
        rs(a3 * t4),
        rs(p8_ref[...] * eye) * flag,
        rs(a1 * t8),
        rs(a2 * t8),
        rs(a3 * t8),
        rs(p12_ref[...] * eye) * flag,
        rs(a1 * t12),
        rs(a2 * t12),
        rs(a3 * t12),
        rs(a4 * t12),
    ]
    o_ref[...] += jnp.concatenate(cols, axis=1)


def _diag_pairs(p1, p2, p3, p4, p8, p12):
    g = N // _BT
    bs = lambda f: pl.BlockSpec((_BT, _BT), f)
    return pl.pallas_call(
        _diag_kernel,
        grid=(g, g),
        in_specs=[bs(lambda i, j: (i, j)),
                  bs(lambda i, j: (i, j)),
                  bs(lambda i, j: (i, j)),
                  bs(lambda i, j: (i, j)),
                  bs(lambda i, j: (j, i)),
                  bs(lambda i, j: (j, i)),
                  bs(lambda i, j: (j, i))],
        out_specs=pl.BlockSpec((_BT, WALK), lambda i, j: (i, 0)),
        out_shape=jax.ShapeDtypeStruct((N, WALK), jnp.float32),
    )(p1, p2, p3, p4, p4, p8, p12)


_RB = 256


def _cdot(a, b):
    return lax.dot_general(a, b, (((1,), (1,)), ((), ())),
                           preferred_element_type=jnp.float32)


def _inproj_kernel(x_ref, dg_ref, wr_ref, br_ref, w1_ref, w2_ref, b_ref,
                   h_ref):
    pe = _cdot(dg_ref[...], wr_ref[...]) + br_ref[...]
    h = _cdot(x_ref[...], w1_ref[...]) + _cdot(pe, w2_ref[...])
    h_ref[...] = h + b_ref[...]


def _input_proj(x, diags, w_rwse, b_rwse, w1, w2, b_in):
    return pl.pallas_call(
        _inproj_kernel,
        grid=(N // _RB,),
        in_specs=[
            pl.BlockSpec((_RB, D_FEAT), lambda i: (i, 0)),
            pl.BlockSpec((_RB, WALK), lambda i: (i, 0)),
            pl.BlockSpec((RWSE_DIM, WALK), lambda i: (0, 0)),
            pl.BlockSpec((1, RWSE_DIM), lambda i: (0, 0)),
            pl.BlockSpec((HID, D_FEAT), lambda i: (0, 0)),
            pl.BlockSpec((HID, RWSE_DIM), lambda i: (0, 0)),
            pl.BlockSpec((1, HID), lambda i: (0, 0)),
        ],
        out_specs=pl.BlockSpec((_RB, HID), lambda i: (i, 0)),
        out_shape=jax.ShapeDtypeStruct((N, HID), jnp.float32),
    )(x, diags, w_rwse, b_rwse, w1, w2, b_in)


def _proj_kernel(h_ref, w_ref, b_ref, o_ref):
    o_ref[...] = _cdot(h_ref[...], w_ref[...]) + b_ref[...]


def _proj(h, wcat, bcat):
    return pl.pallas_call(
        _proj_kernel,
        grid=(N // _RB,),
        in_specs=[
            pl.BlockSpec((_RB, HID), lambda i: (i, 0)),
            pl.BlockSpec((4 * HID, HID), lambda i: (0, 0)),
            pl.BlockSpec((1, 4 * HID), lambda i: (0, 0)),
        ],
        out_specs=pl.BlockSpec((_RB, 4 * HID), lambda i: (i, 0)),
        out_shape=jax.ShapeDtypeStruct((N, 4 * HID), jnp.float32),
    )(h, wcat, bcat)


_BA = 512


def _attn_kernel(q_ref, kv_ref, a_ref, num_ref, den_ref):
    @pl.when(pl.program_id(1) == 0)
    def _():
        num_ref[...] = jnp.zeros_like(num_ref)
        den_ref[...] = jnp.zeros_like(den_ref)

    at = a_ref[...]
    for h in range(HEADS):
        sl = slice(h * DH, (h + 1) * DH)
        ksl = slice(HID + h * DH, HID + (h + 1) * DH)
        vsl = slice(2 * HID + h * DH, 2 * HID + (h + 1) * DH)
        st = _cdot(kv_ref[:, ksl], q_ref[:, sl])
        w = at * jnp.exp(st * 0.25)
        num_ref[:, sl] += lax.dot_general(
            w, kv_ref[:, vsl], (((0,), (0,)), ((), ())),
            preferred_element_type=jnp.float32)
        den_ref[:, h:h + 1] += jnp.sum(w, axis=0)[:, None]


def _attention(qkvs, a):
    g = N // _BA
    return pl.pallas_call(
        _attn_kernel,
        grid=(g, g),
        in_specs=[
            pl.BlockSpec((_BA, 4 * HID), lambda i, j: (i, 0)),
            pl.BlockSpec((_BA, 4 * HID), lambda i, j: (j, 0)),
            pl.BlockSpec((_BA, _BA), lambda i, j: (j, i)),
        ],
        out_specs=[
            pl.BlockSpec((_BA, HID), lambda i, j: (i, 0)),
            pl.BlockSpec((_BA, HEADS), lambda i, j: (i, 0)),
        ],
        out_shape=[
            jax.ShapeDtypeStruct((N, HID), jnp.float32),
            jax.ShapeDtypeStruct((N, HEADS), jnp.float32),
        ],
    )(qkvs, qkvs, a)


def _combine_kernel(pad, h_ref, s_ref, num_ref, den_ref, g_ref, b_ref, o_ref):
    den = den_ref[...]
    denr = jnp.concatenate(
        [jnp.broadcast_to(den[:, h:h + 1], (_RB, DH)) for h in range(HEADS)],
        axis=1)
    conv = num_ref[...] / (denr + 1e-16) + s_ref[:, 3 * HID:4 * HID]
    z = h_ref[...] + conv
    mu = jnp.mean(z, axis=1, keepdims=True)
    zc = z - mu
    var = jnp.mean(zc * zc, axis=1, keepdims=True)
    hn = zc / jnp.sqrt(var + 1e-5) * g_ref[...] + b_ref[...]
    res = jnp.maximum(hn, 0.0)
    if pad:
        res = jnp.concatenate([res, jnp.zeros_like(res)], axis=1)
    o_ref[...] = res


def _combine(h, qkvs, num, den, g, be, pad=False):
    width = 2 * HID if pad else HID
    return pl.pallas_call(
        functools.partial(_combine_kernel, pad),
        grid=(N // _RB,),
        in_specs=[
            pl.BlockSpec((_RB, HID), lambda i: (i, 0)),
            pl.BlockSpec((_RB, 4 * HID), lambda i: (i, 0)),
            pl.BlockSpec((_RB, HID), lambda i: (i, 0)),
            pl.BlockSpec((_RB, HEADS), lambda i: (i, 0)),
            pl.BlockSpec((1, HID), lambda i: (0, 0)),
            pl.BlockSpec((1, HID), lambda i: (0, 0)),
        ],
        out_specs=pl.BlockSpec((_RB, width), lambda i: (i, 0)),
        out_shape=jax.ShapeDtypeStruct((N, width), jnp.float32),
    )(h, qkvs, num, den, g, be)


PAIRS_PER = NPAIRS // 32


@functools.cache
def _pairs_kernel():
    return functools.partial(
        pl.kernel,
        out_type=jax.ShapeDtypeStruct((NPAIRS,), jnp.float32),
        mesh=_sc_mesh(),
        scratch_types=[
            pltpu.VMEM((PAIRS_PER,), jnp.int32),
            pltpu.VMEM((PAIRS_PER,), jnp.int32),
            pltpu.VMEM((PAIRS_PER, 2 * HID), jnp.float32),
            pltpu.VMEM((PAIRS_PER, 2 * HID), jnp.float32),
            pltpu.VMEM((PAIRS_PER,), jnp.float32),
            pltpu.SemaphoreType.DMA,
        ],
        compiler_params=_SC_PARAMS,
    )(_pairs_body)


def _pairs(h, src, dst):
    return _pairs_kernel()(h, src, dst)


def _pairs_body(h_hbm, src_hbm, dst_hbm, out_hbm, s_v, d_v, hs_v, hd_v, res_v,
                sem):
    wid = lax.axis_index("s") * 2 + lax.axis_index("c")
    base = wid * PAIRS_PER
    pltpu.sync_copy(src_hbm.at[pl.ds(base, PAIRS_PER)], s_v)
    pltpu.sync_copy(dst_hbm.at[pl.ds(base, PAIRS_PER)], d_v)
    pltpu.async_copy(h_hbm.at[s_v], hs_v, sem).wait()
    pltpu.async_copy(h_hbm.at[d_v], hd_v, sem).wait()

    def group_body(g, _):
        rows = g * 16 + lax.iota(jnp.int32, 16)
        acc = jnp.zeros((16,), jnp.float32)
        for c in range(HID):
            colv = jnp.full((16,), c, jnp.int32)
            acc = acc + (plsc.load_gather(hs_v, [rows, colv])
                         * plsc.load_gather(hd_v, [rows, colv]))
        res_v[pl.ds(g * 16, 16)] = 1.0 / (1.0 + jnp.exp(-acc))
        return 0

    lax.fori_loop(0, PAIRS_PER // 16, group_body, 0)
    pltpu.sync_copy(res_v, out_hbm.at[pl.ds(base, PAIRS_PER)])


def kernel(x, edge_index, src, dst, W_rwse, b_rwse, W_in, b_in,
           Wq0, bq0, Wk0, bk0, Wv0, bv0, Ws0, bs0, g0, be0,
           Wq1, bq1, Wk1, bk1, Wv1, bv1, Ws1, bs1, g1, be1):
    row = edge_index[0]
    col = edge_index[1]
    a = _build_counts(row, col)
    p = _normalize(a)
    p2 = _mm(p, p)
    p3 = _mm(p2, p)
    p4 = _mm(p2, p2)
    p8 = _mm(p4, p4)
    p12 = _mm(p8, p4)
    diags = _diag_pairs(p, p2, p3, p4, p8, p12)
    h = _input_proj(x, diags, W_rwse, b_rwse.reshape(1, -1),
                    W_in[:, :D_FEAT], W_in[:, D_FEAT:], b_in.reshape(1, -1))
    layers = ((Wq0, bq0, Wk0, bk0, Wv0, bv0, Ws0, bs0, g0, be0),
              (Wq1, bq1, Wk1, bk1, Wv1, bv1, Ws1, bs1, g1, be1))
    for li, (wq, bq, wk, bk, wv, bv, ws, bs, g, be) in enumerate(layers):
        wcat = jnp.concatenate([wq, wk, wv, ws], axis=0)
        bcat = jnp.concatenate([bq, bk, bv, bs]).reshape(1, -1)
        qkvs = _proj(h, wcat, bcat)
        num, den = _attention(qkvs, a)
        h = _combine(h, qkvs, num, den, g.reshape(1, -1), be.reshape(1, -1),
                     pad=(li == len(layers) - 1))
    return _pairs(h, src, dst)


# speedup vs baseline: 10.4215x; 1.1183x over previous
import functools

import jax
import jax.numpy as jnp
from jax import lax
from jax.experimental import pallas as pl
from jax.experimental.pallas import tpu as pltpu
from jax.experimental.pallas import tpu_sc as plsc

N = 2048
E = 32768
D_FEAT = 128
HID = 64
HEADS = 4
DH = 16
WALK = 16
RWSE_DIM = 16
NPAIRS = 4096

_SC_PARAMS = pltpu.CompilerParams(needs_layout_passes=False)


@functools.cache
def _sc_mesh():
    return plsc.VectorSubcoreMesh(core_axis_name="c", subcore_axis_name="s")


ROWS_PER_TILE = 64
COLS_PER_PASS = 1024
ECHUNK = 8192


@functools.cache
def _build_counts_kernel():
    return functools.partial(
        pl.kernel,
        out_type=jax.ShapeDtypeStruct((N, N), jnp.float32),
        mesh=_sc_mesh(),
        scratch_types=[
            pltpu.VMEM((ROWS_PER_TILE, COLS_PER_PASS), jnp.float32),
            pltpu.VMEM((ECHUNK,), jnp.int32),
            pltpu.VMEM((ECHUNK,), jnp.int32),
        ],
        compiler_params=_SC_PARAMS,
    )(_build_counts_body)


def _build_counts(row, col):
    return _build_counts_kernel()(row, col)


def _build_counts_body(row_hbm, col_hbm, out_hbm, acc_v, r_v, c_v):
    wid = lax.axis_index("s") * 2 + lax.axis_index("c")
    r0 = wid * ROWS_PER_TILE
    zeros16 = jnp.zeros((16,), jnp.float32)
    ones16 = jnp.ones((16,), jnp.float32)
    for p in range(N // COLS_PER_PASS):
        c0 = p * COLS_PER_PASS

        def zero_body(i, _):
            r = i // (COLS_PER_PASS // 16)
            c = (i % (COLS_PER_PASS // 16)) * 16
            acc_v[r, pl.ds(c, 16)] = zeros16
            return 0

        lax.fori_loop(0, ROWS_PER_TILE * COLS_PER_PASS // 16, zero_body, 0)

        def chunk_body(ch, _):
            pltpu.sync_copy(row_hbm.at[pl.ds(ch * ECHUNK, ECHUNK)], r_v)
            pltpu.sync_copy(col_hbm.at[pl.ds(ch * ECHUNK, ECHUNK)], c_v)

            def group_body(g, _):
                r16 = r_v[pl.ds(g * 16, 16)]
                c16 = c_v[pl.ds(g * 16, 16)]
                m = ((r16 >= r0) & (r16 < r0 + ROWS_PER_TILE)
                     & (c16 >= c0) & (c16 < c0 + COLS_PER_PASS))
                rr = jnp.where(m, r16 - r0, 0)
                cc = jnp.where(m, c16 - c0, 0)
                plsc.addupdate_scatter(acc_v, [rr, cc], ones16, mask=m)
                return 0

            lax.fori_loop(0, ECHUNK // 16, group_body, 0)
            return 0

        lax.fori_loop(0, E // ECHUNK, chunk_body, 0)
        pltpu.sync_copy(
            acc_v, out_hbm.at[pl.ds(r0, ROWS_PER_TILE), pl.ds(c0, COLS_PER_PASS)])


def _norm_kernel(a_ref, p_ref):
    a = a_ref[...]
    deg = jnp.sum(a, axis=1, keepdims=True)
    p_ref[...] = (a / jnp.maximum(deg, 1.0)).astype(jnp.bfloat16)


def _normalize(a):
    return pl.pallas_call(
        _norm_kernel,
        grid=(8,),
        in_specs=[pl.BlockSpec((N // 8, N), lambda i: (i, 0))],
        out_specs=pl.BlockSpec((N // 8, N), lambda i: (i, 0)),
        out_shape=jax.ShapeDtypeStruct((N, N), jnp.bfloat16),
    )(a)


_MB = 512


def _mm_kernel(a_ref, b_ref, o_ref, acc_ref):
    @pl.when(pl.program_id(2) == 0)
    def _():
        acc_ref[...] = jnp.zeros_like(acc_ref)

    acc_ref[...] += jnp.dot(a_ref[...], b_ref[...],
                            preferred_element_type=jnp.float32)

    @pl.when(pl.program_id(2) == pl.num_programs(2) - 1)
    def _():
        o_ref[...] = acc_ref[...].astype(jnp.bfloat16)


def _mm(a, b):
    g = N // _MB
    return pl.pallas_call(
        _mm_kernel,
        grid=(g, g, g),
        in_specs=[pl.BlockSpec((_MB, _MB), lambda i, j, k: (i, k)),
                  pl.BlockSpec((_MB, _MB), lambda i, j, k: (k, j))],
        out_specs=pl.BlockSpec((_MB, _MB), lambda i, j, k: (i, j)),
        out_shape=jax.ShapeDtypeStruct((N, N), jnp.bfloat16),
        scratch_shapes=[pltpu.VMEM((_MB, _MB), jnp.float32)],
    )(a, b)


_BT = 128


def _diag_kernel(p1_ref, p2_ref, p3_ref, p4a_ref, p4b_ref, p8_ref, p12_ref,
                 o_ref):
    bi = pl.program_id(0)
    bj = pl.program_id(1)

    @pl.when(bj == 0)
    def _():
        o_ref[...] = jnp.zeros_like(o_ref)

    ii = lax.broadcasted_iota(jnp.int32, (_BT, _BT), 0)
    jj = lax.broadcasted_iota(jnp.int32, (_BT, _BT), 1)
    eye = (ii == jj).astype(jnp.float32)
    eye_b = eye.astype(jnp.bfloat16)

    def tr(x):
        return lax.dot_general(x, eye_b, (((0,), (0,)), ((), ())),
                               preferred_element_type=jnp.float32)

    t4 = tr(p4b_ref[...])
    t8 = tr(p8_ref[...])
    t12 = tr(p12_ref[...])
    a1 = p1_ref[...].astype(jnp.float32)
    a2 = p2_ref[...].astype(jnp.float32)
    a3 = p3_ref[...].astype(jnp.float32)
    a4 = p4a_ref[...].astype(jnp.float32)
    flag = jnp.where(bi == bj, 1.0, 0.0)

    def rs(x):
        return jnp.sum(x, axis=1, keepdims=True)

    cols = [
        rs(a1 * eye) * flag,
        rs(a2 * eye) * flag,
        rs(a3 * eye) * flag,
        rs(a4 * eye) * flag,
        rs(a1 * t4),
        rs(a2 * t4),
        rs(a3 * t4),
        rs(p8_ref[...] * eye) * flag,
        rs(a1 * t8),
        rs(a2 * t8),
        rs(a3 * t8),
        rs(p12_ref[...] * eye) * flag,
        rs(a1 * t12),
        rs(a2 * t12),
        rs(a3 * t12),
        rs(a4 * t12),
    ]
    o_ref[...] += jnp.concatenate(cols, axis=1)


def _diag_pairs(p1, p2, p3, p4, p8, p12):
    g = N // _BT
    bs = lambda f: pl.BlockSpec((_BT, _BT), f)
    return pl.pallas_call(
        _diag_kernel,
        grid=(g, g),
        in_specs=[bs(lambda i, j: (i, j)),
                  bs(lambda i, j: (i, j)),
                  bs(lambda i, j: (i, j)),
                  bs(lambda i, j: (i, j)),
                  bs(lambda i, j: (j, i)),
                  bs(lambda i, j: (j, i)),
                  bs(lambda i, j: (j, i))],
        out_specs=pl.BlockSpec((_BT, WALK), lambda i, j: (i, 0)),
        out_shape=jax.ShapeDtypeStruct((N, WALK), jnp.float32),
    )(p1, p2, p3, p4, p4, p8, p12)


_RB = 256


def _cdot(a, b):
    return lax.dot_general(a, b, (((1,), (1,)), ((), ())),
                           preferred_element_type=jnp.float32)


def _inproj_kernel(x_ref, dg_ref, wr_ref, br_ref, w1_ref, w2_ref, b_ref,
                   h_ref):
    pe = _cdot(dg_ref[...], wr_ref[...]) + br_ref[...]
    h = _cdot(x_ref[...], w1_ref[...]) + _cdot(pe, w2_ref[...])
    h_ref[...] = h + b_ref[...]


def _input_proj(x, diags, w_rwse, b_rwse, w1, w2, b_in):
    return pl.pallas_call(
        _inproj_kernel,
        grid=(N // _RB,),
        in_specs=[
            pl.BlockSpec((_RB, D_FEAT), lambda i: (i, 0)),
            pl.BlockSpec((_RB, WALK), lambda i: (i, 0)),
            pl.BlockSpec((RWSE_DIM, WALK), lambda i: (0, 0)),
            pl.BlockSpec((1, RWSE_DIM), lambda i: (0, 0)),
            pl.BlockSpec((HID, D_FEAT), lambda i: (0, 0)),
            pl.BlockSpec((HID, RWSE_DIM), lambda i: (0, 0)),
            pl.BlockSpec((1, HID), lambda i: (0, 0)),
        ],
        out_specs=pl.BlockSpec((_RB, HID), lambda i: (i, 0)),
        out_shape=jax.ShapeDtypeStruct((N, HID), jnp.float32),
    )(x, diags, w_rwse, b_rwse, w1, w2, b_in)


def _proj_kernel(h_ref, w_ref, b_ref, o_ref):
    o_ref[...] = _cdot(h_ref[...], w_ref[...]) + b_ref[...]


def _proj(h, wcat, bcat):
    return pl.pallas_call(
        _proj_kernel,
        grid=(N // _RB,),
        in_specs=[
            pl.BlockSpec((_RB, HID), lambda i: (i, 0)),
            pl.BlockSpec((4 * HID, HID), lambda i: (0, 0)),
            pl.BlockSpec((1, 4 * HID), lambda i: (0, 0)),
        ],
        out_specs=pl.BlockSpec((_RB, 4 * HID), lambda i: (i, 0)),
        out_shape=jax.ShapeDtypeStruct((N, 4 * HID), jnp.float32),
    )(h, wcat, bcat)


_BA = 512


def _attn_kernel(q_ref, kv_ref, a_ref, num_ref, den_ref):
    @pl.when(pl.program_id(1) == 0)
    def _():
        num_ref[...] = jnp.zeros_like(num_ref)
        den_ref[...] = jnp.zeros_like(den_ref)

    at = a_ref[...]
    for h in range(HEADS):
        sl = slice(h * DH, (h + 1) * DH)
        ksl = slice(HID + h * DH, HID + (h + 1) * DH)
        vsl = slice(2 * HID + h * DH, 2 * HID + (h + 1) * DH)
        st = _cdot(kv_ref[:, ksl], q_ref[:, sl])
        w = at * jnp.exp(st * 0.25)
        num_ref[:, sl] += lax.dot_general(
            w, kv_ref[:, vsl], (((0,), (0,)), ((), ())),
            preferred_element_type=jnp.float32)
        den_ref[:, h:h + 1] += jnp.sum(w, axis=0)[:, None]


def _attention(qkvs, a):
    g = N // _BA
    return pl.pallas_call(
        _attn_kernel,
        grid=(g, g),
        in_specs=[
            pl.BlockSpec((_BA, 4 * HID), lambda i, j: (i, 0)),
            pl.BlockSpec((_BA, 4 * HID), lambda i, j: (j, 0)),
            pl.BlockSpec((_BA, _BA), lambda i, j: (j, i)),
        ],
        out_specs=[
            pl.BlockSpec((_BA, HID), lambda i, j: (i, 0)),
            pl.BlockSpec((_BA, HEADS), lambda i, j: (i, 0)),
        ],
        out_shape=[
            jax.ShapeDtypeStruct((N, HID), jnp.float32),
            jax.ShapeDtypeStruct((N, HEADS), jnp.float32),
        ],
    )(qkvs, qkvs, a)


def _combine_kernel(pad, h_ref, s_ref, num_ref, den_ref, g_ref, b_ref, o_ref):
    den = den_ref[...]
    denr = jnp.concatenate(
        [jnp.broadcast_to(den[:, h:h + 1], (_RB, DH)) for h in range(HEADS)],
        axis=1)
    conv = num_ref[...] / (denr + 1e-16) + s_ref[:, 3 * HID:4 * HID]
    z = h_ref[...] + conv
    mu = jnp.mean(z, axis=1, keepdims=True)
    zc = z - mu
    var = jnp.mean(zc * zc, axis=1, keepdims=True)
    hn = zc / jnp.sqrt(var + 1e-5) * g_ref[...] + b_ref[...]
    res = jnp.maximum(hn, 0.0)
    if pad:
        res = jnp.concatenate([res, jnp.zeros_like(res)], axis=1)
    o_ref[...] = res


def _combine(h, qkvs, num, den, g, be, pad=False):
    width = 2 * HID if pad else HID
    return pl.pallas_call(
        functools.partial(_combine_kernel, pad),
        grid=(N // _RB,),
        in_specs=[
            pl.BlockSpec((_RB, HID), lambda i: (i, 0)),
            pl.BlockSpec((_RB, 4 * HID), lambda i: (i, 0)),
            pl.BlockSpec((_RB, HID), lambda i: (i, 0)),
            pl.BlockSpec((_RB, HEADS), lambda i: (i, 0)),
            pl.BlockSpec((1, HID), lambda i: (0, 0)),
            pl.BlockSpec((1, HID), lambda i: (0, 0)),
        ],
        out_specs=pl.BlockSpec((_RB, width), lambda i: (i, 0)),
        out_shape=jax.ShapeDtypeStruct((N, width), jnp.float32),
    )(h, qkvs, num, den, g, be)


PAIRS_PER = NPAIRS // 32


@functools.cache
def _pairs_kernel():
    return functools.partial(
        pl.kernel,
        out_type=jax.ShapeDtypeStruct((NPAIRS,), jnp.float32),
        mesh=_sc_mesh(),
        scratch_types=[
            pltpu.VMEM((PAIRS_PER,), jnp.int32),
            pltpu.VMEM((PAIRS_PER,), jnp.int32),
            pltpu.VMEM((PAIRS_PER, 2 * HID), jnp.float32),
            pltpu.VMEM((PAIRS_PER, 2 * HID), jnp.float32),
            pltpu.VMEM((PAIRS_PER,), jnp.float32),
            pltpu.SemaphoreType.DMA,
        ],
        compiler_params=_SC_PARAMS,
    )(_pairs_body)


def _pairs(h, src, dst):
    return _pairs_kernel()(h, src, dst)


def _pairs_body(h_hbm, src_hbm, dst_hbm, out_hbm, s_v, d_v, hs_v, hd_v, res_v,
                sem):
    wid = lax.axis_index("s") * 2 + lax.axis_index("c")
    base = wid * PAIRS_PER
    pltpu.sync_copy(src_hbm.at[pl.ds(base, PAIRS_PER)], s_v)
    pltpu.sync_copy(dst_hbm.at[pl.ds(base, PAIRS_PER)], d_v)
    pltpu.async_copy(h_hbm.at[s_v], hs_v, sem).wait()
    pltpu.async_copy(h_hbm.at[d_v], hd_v, sem).wait()

    def group_body(g, _):
        rows = g * 16 + lax.iota(jnp.int32, 16)
        acc = jnp.zeros((16,), jnp.float32)
        for c in range(HID):
            colv = jnp.full((16,), c, jnp.int32)
            acc = acc + (plsc.load_gather(hs_v, [rows, colv])
                         * plsc.load_gather(hd_v, [rows, colv]))
        res_v[pl.ds(g * 16, 16)] = 1.0 / (1.0 + jnp.exp(-acc))
        return 0

    lax.fori_loop(0, PAIRS_PER // 16, group_body, 0)
    pltpu.sync_copy(res_v, out_hbm.at[pl.ds(base, PAIRS_PER)])


def kernel(x, edge_index, src, dst, W_rwse, b_rwse, W_in, b_in,
           Wq0, bq0, Wk0, bk0, Wv0, bv0, Ws0, bs0, g0, be0,
           Wq1, bq1, Wk1, bk1, Wv1, bv1, Ws1, bs1, g1, be1):
    row = edge_index[0]
    col = edge_index[1]
    a = _build_counts(row, col)
    p = _normalize(a)
    p2 = _mm(p, p)
    p3 = _mm(p2, p)
    p4 = _mm(p2, p2)
    p8 = _mm(p4, p4)
    p12 = _mm(p8, p4)
    diags = _diag_pairs(p, p2, p3, p4, p8, p12)
    h = _input_proj(x, diags, W_rwse, b_rwse.reshape(1, -1),
                    W_in[:, :D_FEAT], W_in[:, D_FEAT:], b_in.reshape(1, -1))
    layers = ((Wq0, bq0, Wk0, bk0, Wv0, bv0, Ws0, bs0, g0, be0),
              (Wq1, bq1, Wk1, bk1, Wv1, bv1, Ws1, bs1, g1, be1))
    for li, (wq, bq, wk, bk, wv, bv, ws, bs, g, be) in enumerate(layers):
        wcat = jnp.concatenate([wq, wk, wv, ws], axis=0)
        bcat = jnp.concatenate([bq, bk, bv, bs]).reshape(1, -1)
        qkvs = _proj(h, wcat, bcat)
        num, den = _attention(qkvs, a)
        h = _combine(h, qkvs, num, den, g.reshape(1, -1), be.reshape(1, -1),
                     pad=(li == len(layers) - 1))
    return _pairs(h, src, dst)


# speedup vs baseline: 12.9216x; 1.2399x over previous
import functools

import jax
import jax.numpy as jnp
from jax import lax
from jax.experimental import pallas as pl
from jax.experimental.pallas import tpu as pltpu
from jax.experimental.pallas import tpu_sc as plsc

N = 2048
E = 32768
D_FEAT = 128
HID = 64
HEADS = 4
DH = 16
WALK = 16
RWSE_DIM = 16
NPAIRS = 4096

_SC_PARAMS = pltpu.CompilerParams(needs_layout_passes=False)


@functools.cache
def _sc_mesh():
    return plsc.VectorSubcoreMesh(core_axis_name="c", subcore_axis_name="s")


ROWS_PER_TILE = 64
COLS_PER_PASS = 1024
ECHUNK = 8192


@functools.cache
def _build_counts_kernel():
    return functools.partial(
        pl.kernel,
        out_type=jax.ShapeDtypeStruct((N, N), jnp.float32),
        mesh=_sc_mesh(),
        scratch_types=[
            pltpu.VMEM((ROWS_PER_TILE, COLS_PER_PASS), jnp.float32),
            pltpu.VMEM((ECHUNK,), jnp.int32),
            pltpu.VMEM((ECHUNK,), jnp.int32),
        ],
        compiler_params=_SC_PARAMS,
    )(_build_counts_body)


def _build_counts(row, col):
    return _build_counts_kernel()(row, col)


def _build_counts_body(row_hbm, col_hbm, out_hbm, acc_v, r_v, c_v):
    wid = lax.axis_index("s") * 2 + lax.axis_index("c")
    r0 = wid * ROWS_PER_TILE
    zeros16 = jnp.zeros((16,), jnp.float32)
    ones16 = jnp.ones((16,), jnp.float32)
    for p in range(N // COLS_PER_PASS):
        c0 = p * COLS_PER_PASS

        def zero_body(i, _):
            r = i // 8
            cbase = (i % 8) * 128
            for u in range(8):
                acc_v[r, pl.ds(cbase + u * 16, 16)] = zeros16
            return 0

        lax.fori_loop(0, ROWS_PER_TILE * COLS_PER_PASS // (16 * 8),
                      zero_body, 0)

        def chunk_body(ch, _):
            pltpu.sync_copy(row_hbm.at[pl.ds(ch * ECHUNK, ECHUNK)], r_v)
            pltpu.sync_copy(col_hbm.at[pl.ds(ch * ECHUNK, ECHUNK)], c_v)

            def group_body(g, _):
                for u in range(4):
                    o = (g * 4 + u) * 16
                    r16 = r_v[pl.ds(o, 16)]
                    c16 = c_v[pl.ds(o, 16)]
                    m = ((r16 >= r0) & (r16 < r0 + ROWS_PER_TILE)
                         & (c16 >= c0) & (c16 < c0 + COLS_PER_PASS))
                    rr = jnp.where(m, r16 - r0, 0)
                    cc = jnp.where(m, c16 - c0, 0)
                    plsc.addupdate_scatter(acc_v, [rr, cc], ones16, mask=m)
                return 0

            lax.fori_loop(0, ECHUNK // (16 * 4), group_body, 0)
            return 0

        lax.fori_loop(0, E // ECHUNK, chunk_body, 0)
        pltpu.sync_copy(
            acc_v, out_hbm.at[pl.ds(r0, ROWS_PER_TILE), pl.ds(c0, COLS_PER_PASS)])


def _norm_kernel(a_ref, p_ref):
    a = a_ref[...]
    deg = jnp.sum(a, axis=1, keepdims=True)
    p_ref[...] = (a / jnp.maximum(deg, 1.0)).astype(jnp.bfloat16)


def _normalize(a):
    return pl.pallas_call(
        _norm_kernel,
        grid=(8,),
        in_specs=[pl.BlockSpec((N // 8, N), lambda i: (i, 0))],
        out_specs=pl.BlockSpec((N // 8, N), lambda i: (i, 0)),
        out_shape=jax.ShapeDtypeStruct((N, N), jnp.bfloat16),
    )(a)


_MB = 512


def _mm_kernel(a_ref, b_ref, o_ref, acc_ref):
    @pl.when(pl.program_id(2) == 0)
    def _():
        acc_ref[...] = jnp.zeros_like(acc_ref)

    acc_ref[...] += jnp.dot(a_ref[...], b_ref[...],
                            preferred_element_type=jnp.float32)

    @pl.when(pl.program_id(2) == pl.num_programs(2) - 1)
    def _():
        o_ref[...] = acc_ref[...].astype(jnp.bfloat16)


def _mm(a, b):
    g = N // _MB
    return pl.pallas_call(
        _mm_kernel,
        grid=(g, g, g),
        in_specs=[pl.BlockSpec((_MB, _MB), lambda i, j, k: (i, k)),
                  pl.BlockSpec((_MB, _MB), lambda i, j, k: (k, j))],
        out_specs=pl.BlockSpec((_MB, _MB), lambda i, j, k: (i, j)),
        out_shape=jax.ShapeDtypeStruct((N, N), jnp.bfloat16),
        scratch_shapes=[pltpu.VMEM((_MB, _MB), jnp.float32)],
    )(a, b)


_BT = 128
_DK = 512


def _diag_kernel(p1_ref, p2_ref, p3_ref, p4r_ref, p4c_ref, p8c_ref, p12c_ref,
                 o_ref, acc_ref):
    bi = pl.program_id(0)
    kk = pl.program_id(1)
    nk = pl.num_programs(1)

    @pl.when(kk == 0)
    def _():
        o_ref[...] = jnp.zeros_like(o_ref)
        acc_ref[...] = jnp.zeros_like(acc_ref)

    a_chunks = [p1_ref[...], p2_ref[...], p3_ref[...], p4r_ref[...]]
    b4 = p4c_ref[...]
    b8 = p8c_ref[...]
    b12 = p12c_ref[...]

    def dot(a, b):
        return jnp.dot(a, b, preferred_element_type=jnp.float32)

    for t, a in enumerate(a_chunks[:3]):
        acc_ref[t * _BT:(t + 1) * _BT, :] += dot(a, b4)
        acc_ref[(3 + t) * _BT:(4 + t) * _BT, :] += dot(a, b8)
    for t, a in enumerate(a_chunks):
        acc_ref[(6 + t) * _BT:(7 + t) * _BT, :] += dot(a, b12)

    ii = lax.broadcasted_iota(jnp.int32, (_BT, _BT), 0)
    jj = lax.broadcasted_iota(jnp.int32, (_BT, _BT), 1)
    eye = (ii == jj).astype(jnp.float32)

    @pl.when(kk == bi // 4)
    def _():
        off = (bi % 4) * _BT
        iw = lax.broadcasted_iota(jnp.int32, (_BT, _DK), 0)
        jw = lax.broadcasted_iota(jnp.int32, (_BT, _DK), 1)
        mask_a = (jw == iw + off).astype(jnp.float32)
        it = lax.broadcasted_iota(jnp.int32, (_DK, _BT), 0)
        jt = lax.broadcasted_iota(jnp.int32, (_DK, _BT), 1)
        mask_b = (it == jt + off).astype(jnp.float32)
        z = jnp.zeros((_BT, 1), jnp.float32)

        def rs_a(x):
            return jnp.sum(x.astype(jnp.float32) * mask_a, axis=1,
                           keepdims=True)

        def rs_b(x):
            return jnp.sum(x.astype(jnp.float32) * mask_b, axis=0)[:, None]

        o_ref[...] += jnp.concatenate(
            [rs_a(a_chunks[0]), rs_a(a_chunks[1]), rs_a(a_chunks[2]),
             rs_a(a_chunks[3]), z, z, z, rs_b(b8), z, z, z, rs_b(b12),
             z, z, z, z], axis=1)

    @pl.when(kk == nk - 1)
    def _():
        acc = acc_ref[...]

        def dg(t):
            sub = acc[t * _BT:(t + 1) * _BT, :]
            return jnp.sum(sub * eye, axis=1, keepdims=True)

        z = jnp.zeros((_BT, 1), jnp.float32)
        o_ref[...] += jnp.concatenate(
            [z, z, z, z, dg(0), dg(1), dg(2), z, dg(3), dg(4), dg(5), z,
             dg(6), dg(7), dg(8), dg(9)], axis=1)


def _diag_pairs(p1, p2, p3, p4, p8, p12):
    row = pl.BlockSpec((_BT, _DK), lambda i, k: (i, k))
    col = pl.BlockSpec((_DK, _BT), lambda i, k: (k, i))
    return pl.pallas_call(
        _diag_kernel,
        grid=(N // _BT, N // _DK),
        in_specs=[row, row, row, row, col, col, col],
        out_specs=pl.BlockSpec((_BT, WALK), lambda i, k: (i, 0)),
        out_shape=jax.ShapeDtypeStruct((N, WALK), jnp.float32),
        scratch_shapes=[pltpu.VMEM((10 * _BT, _BT), jnp.float32)],
    )(p1, p2, p3, p4, p4, p8, p12)


_RB = 256


def _cdot(a, b):
    return lax.dot_general(a, b, (((1,), (1,)), ((), ())),
                           preferred_element_type=jnp.float32)


def _inproj_kernel(x_ref, dg_ref, wr_ref, br_ref, w1_ref, w2_ref, b_ref,
                   h_ref):
    pe = _cdot(dg_ref[...], wr_ref[...]) + br_ref[...]
    h = _cdot(x_ref[...], w1_ref[...]) + _cdot(pe, w2_ref[...])
    h_ref[...] = h + b_ref[...]


def _input_proj(x, diags, w_rwse, b_rwse, w1, w2, b_in):
    return pl.pallas_call(
        _inproj_kernel,
        grid=(N // _RB,),
        in_specs=[
            pl.BlockSpec((_RB, D_FEAT), lambda i: (i, 0)),
            pl.BlockSpec((_RB, WALK), lambda i: (i, 0)),
            pl.BlockSpec((RWSE_DIM, WALK), lambda i: (0, 0)),
            pl.BlockSpec((1, RWSE_DIM), lambda i: (0, 0)),
            pl.BlockSpec((HID, D_FEAT), lambda i: (0, 0)),
            pl.BlockSpec((HID, RWSE_DIM), lambda i: (0, 0)),
            pl.BlockSpec((1, HID), lambda i: (0, 0)),
        ],
        out_specs=pl.BlockSpec((_RB, HID), lambda i: (i, 0)),
        out_shape=jax.ShapeDtypeStruct((N, HID), jnp.float32),
    )(x, diags, w_rwse, b_rwse, w1, w2, b_in)


def _proj_kernel(h_ref, w_ref, b_ref, o_ref):
    o_ref[...] = _cdot(h_ref[...], w_ref[...]) + b_ref[...]


def _proj(h, wcat, bcat):
    return pl.pallas_call(
        _proj_kernel,
        grid=(N // _RB,),
        in_specs=[
            pl.BlockSpec((_RB, HID), lambda i: (i, 0)),
            pl.BlockSpec((4 * HID, HID), lambda i: (0, 0)),
            pl.BlockSpec((1, 4 * HID), lambda i: (0, 0)),
        ],
        out_specs=pl.BlockSpec((_RB, 4 * HID), lambda i: (i, 0)),
        out_shape=jax.ShapeDtypeStruct((N, 4 * HID), jnp.float32),
    )(h, wcat, bcat)


_BA = 512


def _attn_kernel(q_ref, kv_ref, a_ref, num_ref, den_ref):
    @pl.when(pl.program_id(1) == 0)
    def _():
        num_ref[...] = jnp.zeros_like(num_ref)
        den_ref[...] = jnp.zeros_like(den_ref)

    at = a_ref[...]
    for h in range(HEADS):
        sl = slice(h * DH, (h + 1) * DH)
        ksl = slice(HID + h * DH, HID + (h + 1) * DH)
        vsl = slice(2 * HID + h * DH, 2 * HID + (h + 1) * DH)
        st = _cdot(kv_ref[:, ksl], q_ref[:, sl])
        w = at * jnp.exp(st * 0.25)
        num_ref[:, sl] += lax.dot_general(
            w, kv_ref[:, vsl], (((0,), (0,)), ((), ())),
            preferred_element_type=jnp.float32)
        den_ref[:, h:h + 1] += jnp.sum(w, axis=0)[:, None]


def _attention(qkvs, a):
    g = N // _BA
    return pl.pallas_call(
        _attn_kernel,
        grid=(g, g),
        in_specs=[
            pl.BlockSpec((_BA, 4 * HID), lambda i, j: (i, 0)),
            pl.BlockSpec((_BA, 4 * HID), lambda i, j: (j, 0)),
            pl.BlockSpec((_BA, _BA), lambda i, j: (j, i)),
        ],
        out_specs=[
            pl.BlockSpec((_BA, HID), lambda i, j: (i, 0)),
            pl.BlockSpec((_BA, HEADS), lambda i, j: (i, 0)),
        ],
        out_shape=[
            jax.ShapeDtypeStruct((N, HID), jnp.float32),
            jax.ShapeDtypeStruct((N, HEADS), jnp.float32),
        ],
    )(qkvs, qkvs, a)


def _combine_kernel(pad, h_ref, s_ref, num_ref, den_ref, g_ref, b_ref, o_ref):
    den = den_ref[...]
    denr = jnp.concatenate(
        [jnp.broadcast_to(den[:, h:h + 1], (_RB, DH)) for h in range(HEADS)],
        axis=1)
    conv = num_ref[...] / (denr + 1e-16) + s_ref[:, 3 * HID:4 * HID]
    z = h_ref[...] + conv
    mu = jnp.mean(z, axis=1, keepdims=True)
    zc = z - mu
    var = jnp.mean(zc * zc, axis=1, keepdims=True)
    hn = zc / jnp.sqrt(var + 1e-5) * g_ref[...] + b_ref[...]
    res = jnp.maximum(hn, 0.0)
    if pad:
        res = jnp.concatenate([res, jnp.zeros_like(res)], axis=1)
    o_ref[...] = res


def _combine(h, qkvs, num, den, g, be, pad=False):
    width = 2 * HID if pad else HID
    return pl.pallas_call(
        functools.partial(_combine_kernel, pad),
        grid=(N // _RB,),
        in_specs=[
            pl.BlockSpec((_RB, HID), lambda i: (i, 0)),
            pl.BlockSpec((_RB, 4 * HID), lambda i: (i, 0)),
            pl.BlockSpec((_RB, HID), lambda i: (i, 0)),
            pl.BlockSpec((_RB, HEADS), lambda i: (i, 0)),
            pl.BlockSpec((1, HID), lambda i: (0, 0)),
            pl.BlockSpec((1, HID), lambda i: (0, 0)),
        ],
        out_specs=pl.BlockSpec((_RB, width), lambda i: (i, 0)),
        out_shape=jax.ShapeDtypeStruct((N, width), jnp.float32),
    )(h, qkvs, num, den, g, be)


PAIRS_PER = NPAIRS // 32


@functools.cache
def _pairs_kernel():
    return functools.partial(
        pl.kernel,
        out_type=jax.ShapeDtypeStruct((NPAIRS,), jnp.float32),
        mesh=_sc_mesh(),
        scratch_types=[
            pltpu.VMEM((PAIRS_PER,), jnp.int32),
            pltpu.VMEM((PAIRS_PER,), jnp.int32),
            pltpu.VMEM((PAIRS_PER, 2 * HID), jnp.float32),
            pltpu.VMEM((PAIRS_PER, 2 * HID), jnp.float32),
            pltpu.VMEM((PAIRS_PER,), jnp.float32),
            pltpu.SemaphoreType.DMA,
        ],
        compiler_params=_SC_PARAMS,
    )(_pairs_body)


def _pairs(h, src, dst):
    return _pairs_kernel()(h, src, dst)


def _pairs_body(h_hbm, src_hbm, dst_hbm, out_hbm, s_v, d_v, hs_v, hd_v, res_v,
                sem):
    wid = lax.axis_index("s") * 2 + lax.axis_index("c")
    base = wid * PAIRS_PER
    pltpu.sync_copy(src_hbm.at[pl.ds(base, PAIRS_PER)], s_v)
    pltpu.sync_copy(dst_hbm.at[pl.ds(base, PAIRS_PER)], d_v)
    pltpu.async_copy(h_hbm.at[s_v], hs_v, sem).wait()
    pltpu.async_copy(h_hbm.at[d_v], hd_v, sem).wait()

    def group_body(g, _):
        rows = g * 16 + lax.iota(jnp.int32, 16)
        acc = jnp.zeros((16,), jnp.float32)
        for c in range(HID):
            colv = jnp.full((16,), c, jnp.int32)
            acc = acc + (plsc.load_gather(hs_v, [rows, colv])
                         * plsc.load_gather(hd_v, [rows, colv]))
        res_v[pl.ds(g * 16, 16)] = 1.0 / (1.0 + jnp.exp(-acc))
        return 0

    lax.fori_loop(0, PAIRS_PER // 16, group_body, 0)
    pltpu.sync_copy(res_v, out_hbm.at[pl.ds(base, PAIRS_PER)])


def kernel(x, edge_index, src, dst, W_rwse, b_rwse, W_in, b_in,
           Wq0, bq0, Wk0, bk0, Wv0, bv0, Ws0, bs0, g0, be0,
           Wq1, bq1, Wk1, bk1, Wv1, bv1, Ws1, bs1, g1, be1):
    row = edge_index[0]
    col = edge_index[1]
    a = _build_counts(row, col)
    p = _normalize(a)
    p2 = _mm(p, p)
    p3 = _mm(p2, p)
    p4 = _mm(p2, p2)
    p8 = _mm(p4, p4)
    p12 = _mm(p8, p4)
    diags = _diag_pairs(p, p2, p3, p4, p8, p12)
    h = _input_proj(x, diags, W_rwse, b_rwse.reshape(1, -1),
                    W_in[:, :D_FEAT], W_in[:, D_FEAT:], b_in.reshape(1, -1))
    layers = ((Wq0, bq0, Wk0, bk0, Wv0, bv0, Ws0, bs0, g0, be0),
              (Wq1, bq1, Wk1, bk1, Wv1, bv1, Ws1, bs1, g1, be1))
    for li, (wq, bq, wk, bk, wv, bv, ws, bs, g, be) in enumerate(layers):
        wcat = jnp.concatenate([wq, wk, wv, ws], axis=0)
        bcat = jnp.concatenate([bq, bk, bv, bs]).reshape(1, -1)
        qkvs = _proj(h, wcat, bcat)
        num, den = _attention(qkvs, a)
        h = _combine(h, qkvs, num, den, g.reshape(1, -1), be.reshape(1, -1),
                     pad=(li == len(layers) - 1))
    return _pairs(h, src, dst)


# speedup vs baseline: 14.0420x; 1.0867x over previous
import functools

import jax
import jax.numpy as jnp
from jax import lax
from jax.experimental import pallas as pl
from jax.experimental.pallas import tpu as pltpu
from jax.experimental.pallas import tpu_sc as plsc

N = 2048
E = 32768
D_FEAT = 128
HID = 64
HEADS = 4
DH = 16
WALK = 16
RWSE_DIM = 16
NPAIRS = 4096

_SC_PARAMS = pltpu.CompilerParams(needs_layout_passes=False)


@functools.cache
def _sc_mesh():
    return plsc.VectorSubcoreMesh(core_axis_name="c", subcore_axis_name="s")


ROWS_PER_TILE = 64
COLS_PER_PASS = 1024
ECHUNK = 8192


@functools.cache
def _build_counts_kernel():
    return functools.partial(
        pl.kernel,
        out_type=jax.ShapeDtypeStruct((N, N), jnp.float32),
        mesh=_sc_mesh(),
        scratch_types=[
            pltpu.VMEM((ROWS_PER_TILE, COLS_PER_PASS), jnp.float32),
            pltpu.VMEM((ECHUNK,), jnp.int32),
            pltpu.VMEM((ECHUNK,), jnp.int32),
        ],
        compiler_params=_SC_PARAMS,
    )(_build_counts_body)


def _build_counts(row, col):
    return _build_counts_kernel()(row, col)


def _build_counts_body(row_hbm, col_hbm, out_hbm, acc_v, r_v, c_v):
    wid = lax.axis_index("s") * 2 + lax.axis_index("c")
    r0 = wid * ROWS_PER_TILE
    zeros16 = jnp.zeros((16,), jnp.float32)
    ones16 = jnp.ones((16,), jnp.float32)
    for p in range(N // COLS_PER_PASS):
        c0 = p * COLS_PER_PASS

        def zero_body(i, _):
            r = i // 8
            cbase = (i % 8) * 128
            for u in range(8):
                acc_v[r, pl.ds(cbase + u * 16, 16)] = zeros16
            return 0

        lax.fori_loop(0, ROWS_PER_TILE * COLS_PER_PASS // (16 * 8),
                      zero_body, 0)

        def chunk_body(ch, _):
            pltpu.sync_copy(row_hbm.at[pl.ds(ch * ECHUNK, ECHUNK)], r_v)
            pltpu.sync_copy(col_hbm.at[pl.ds(ch * ECHUNK, ECHUNK)], c_v)

            def group_body(g, _):
                for u in range(4):
                    o = (g * 4 + u) * 16
                    r16 = r_v[pl.ds(o, 16)]
                    c16 = c_v[pl.ds(o, 16)]
                    m = ((r16 >= r0) & (r16 < r0 + ROWS_PER_TILE)
                         & (c16 >= c0) & (c16 < c0 + COLS_PER_PASS))
                    rr = jnp.where(m, r16 - r0, 0)
                    cc = jnp.where(m, c16 - c0, 0)
                    plsc.addupdate_scatter(acc_v, [rr, cc], ones16, mask=m)
                return 0

            lax.fori_loop(0, ECHUNK // (16 * 4), group_body, 0)
            return 0

        lax.fori_loop(0, E // ECHUNK, chunk_body, 0)
        pltpu.sync_copy(
            acc_v, out_hbm.at[pl.ds(r0, ROWS_PER_TILE), pl.ds(c0, COLS_PER_PASS)])


def _norm_kernel(a_ref, p_ref):
    a = a_ref[...]
    deg = jnp.sum(a, axis=1, keepdims=True)
    p_ref[...] = (a / jnp.maximum(deg, 1.0)).astype(jnp.bfloat16)


def _normalize(a):
    return pl.pallas_call(
        _norm_kernel,
        grid=(8,),
        in_specs=[pl.BlockSpec((N // 8, N), lambda i: (i, 0))],
        out_specs=pl.BlockSpec((N // 8, N), lambda i: (i, 0)),
        out_shape=jax.ShapeDtypeStruct((N, N), jnp.bfloat16),
    )(a)


_MB = 512


def _mm_kernel(a_ref, b_ref, o_ref, acc_ref):
    @pl.when(pl.program_id(2) == 0)
    def _():
        acc_ref[...] = jnp.zeros_like(acc_ref)

    acc_ref[...] += jnp.dot(a_ref[...], b_ref[...],
                            preferred_element_type=jnp.float32)

    @pl.when(pl.program_id(2) == pl.num_programs(2) - 1)
    def _():
        o_ref[...] = acc_ref[...].astype(jnp.bfloat16)


def _mm(a, b):
    g = N // _MB
    return pl.pallas_call(
        _mm_kernel,
        grid=(g, g, g),
        in_specs=[pl.BlockSpec((_MB, _MB), lambda i, j, k: (i, k)),
                  pl.BlockSpec((_MB, _MB), lambda i, j, k: (k, j))],
        out_specs=pl.BlockSpec((_MB, _MB), lambda i, j, k: (i, j)),
        out_shape=jax.ShapeDtypeStruct((N, N), jnp.bfloat16),
        scratch_shapes=[pltpu.VMEM((_MB, _MB), jnp.float32)],
    )(a, b)


def _mm2_kernel(a_ref, b1_ref, b2_ref, o1_ref, o2_ref, acc1_ref, acc2_ref):
    @pl.when(pl.program_id(2) == 0)
    def _():
        acc1_ref[...] = jnp.zeros_like(acc1_ref)
        acc2_ref[...] = jnp.zeros_like(acc2_ref)

    a = a_ref[...]
    acc1_ref[...] += jnp.dot(a, b1_ref[...],
                             preferred_element_type=jnp.float32)
    acc2_ref[...] += jnp.dot(a, b2_ref[...],
                             preferred_element_type=jnp.float32)

    @pl.when(pl.program_id(2) == pl.num_programs(2) - 1)
    def _():
        o1_ref[...] = acc1_ref[...].astype(jnp.bfloat16)
        o2_ref[...] = acc2_ref[...].astype(jnp.bfloat16)


def _mm2(a, b1, b2):
    g = N // _MB
    sd = jax.ShapeDtypeStruct((N, N), jnp.bfloat16)
    return pl.pallas_call(
        _mm2_kernel,
        grid=(g, g, g),
        in_specs=[pl.BlockSpec((_MB, _MB), lambda i, j, k: (i, k)),
                  pl.BlockSpec((_MB, _MB), lambda i, j, k: (k, j)),
                  pl.BlockSpec((_MB, _MB), lambda i, j, k: (k, j))],
        out_specs=[pl.BlockSpec((_MB, _MB), lambda i, j, k: (i, j)),
                   pl.BlockSpec((_MB, _MB), lambda i, j, k: (i, j))],
        out_shape=[sd, sd],
        scratch_shapes=[pltpu.VMEM((_MB, _MB), jnp.float32),
                        pltpu.VMEM((_MB, _MB), jnp.float32)],
    )(a, b1, b2)


_BT = 128
_DK = 512


def _diag_kernel(p1_ref, p2_ref, p3_ref, p4r_ref, p4c_ref, p8c_ref, p12c_ref,
                 o_ref, acc_ref):
    bi = pl.program_id(0)
    kk = pl.program_id(1)
    nk = pl.num_programs(1)

    @pl.when(kk == 0)
    def _():
        o_ref[...] = jnp.zeros_like(o_ref)
        acc_ref[...] = jnp.zeros_like(acc_ref)

    a_chunks = [p1_ref[...], p2_ref[...], p3_ref[...], p4r_ref[...]]
    b4 = p4c_ref[...]
    b8 = p8c_ref[...]
    b12 = p12c_ref[...]

    def dot(a, b):
        return jnp.dot(a, b, preferred_element_type=jnp.float32)

    for t, a in enumerate(a_chunks[:3]):
        acc_ref[t * _BT:(t + 1) * _BT, :] += dot(a, b4)
        acc_ref[(3 + t) * _BT:(4 + t) * _BT, :] += dot(a, b8)
    for t, a in enumerate(a_chunks):
        acc_ref[(6 + t) * _BT:(7 + t) * _BT, :] += dot(a, b12)

    ii = lax.broadcasted_iota(jnp.int32, (_BT, _BT), 0)
    jj = lax.broadcasted_iota(jnp.int32, (_BT, _BT), 1)
    eye = (ii == jj).astype(jnp.float32)

    @pl.when(kk == bi // 4)
    def _():
        off = (bi % 4) * _BT
        iw = lax.broadcasted_iota(jnp.int32, (_BT, _DK), 0)
        jw = lax.broadcasted_iota(jnp.int32, (_BT, _DK), 1)
        mask_a = (jw == iw + off).astype(jnp.float32)
        it = lax.broadcasted_iota(jnp.int32, (_DK, _BT), 0)
        jt = lax.broadcasted_iota(jnp.int32, (_DK, _BT), 1)
        mask_b = (it == jt + off).astype(jnp.float32)
        z = jnp.zeros((_BT, 1), jnp.float32)

        def rs_a(x):
            return jnp.sum(x.astype(jnp.float32) * mask_a, axis=1,
                           keepdims=True)

        def rs_b(x):
            return jnp.sum(x.astype(jnp.float32) * mask_b, axis=0)[:, None]

        o_ref[...] += jnp.concatenate(
            [rs_a(a_chunks[0]), rs_a(a_chunks[1]), rs_a(a_chunks[2]),
             rs_a(a_chunks[3]), z, z, z, rs_b(b8), z, z, z, rs_b(b12),
             z, z, z, z], axis=1)

    @pl.when(kk == nk - 1)
    def _():
        acc = acc_ref[...]

        def dg(t):
            sub = acc[t * _BT:(t + 1) * _BT, :]
            return jnp.sum(sub * eye, axis=1, keepdims=True)

        z = jnp.zeros((_BT, 1), jnp.float32)
        o_ref[...] += jnp.concatenate(
            [z, z, z, z, dg(0), dg(1), dg(2), z, dg(3), dg(4), dg(5), z,
             dg(6), dg(7), dg(8), dg(9)], axis=1)


def _diag_pairs(p1, p2, p3, p4, p8, p12):
    row = pl.BlockSpec((_BT, _DK), lambda i, k: (i, k))
    col = pl.BlockSpec((_DK, _BT), lambda i, k: (k, i))
    return pl.pallas_call(
        _diag_kernel,
        grid=(N // _BT, N // _DK),
        in_specs=[row, row, row, row, col, col, col],
        out_specs=pl.BlockSpec((_BT, WALK), lambda i, k: (i, 0)),
        out_shape=jax.ShapeDtypeStruct((N, WALK), jnp.float32),
        scratch_shapes=[pltpu.VMEM((10 * _BT, _BT), jnp.float32)],
    )(p1, p2, p3, p4, p4, p8, p12)


_RB = 256


def _cdot(a, b):
    return lax.dot_general(a, b, (((1,), (1,)), ((), ())),
                           preferred_element_type=jnp.float32)


def _inproj_kernel(x_ref, dg_ref, wr_ref, br_ref, w1_ref, w2_ref, b_ref,
                   wc_ref, bc_ref, h_ref, q_ref):
    pe = _cdot(dg_ref[...], wr_ref[...]) + br_ref[...]
    h = _cdot(x_ref[...], w1_ref[...]) + _cdot(pe, w2_ref[...]) + b_ref[...]
    h_ref[...] = h
    q_ref[...] = _cdot(h, wc_ref[...]) + bc_ref[...]


def _input_proj(x, diags, w_rwse, b_rwse, w1, w2, b_in, wcat, bcat):
    return pl.pallas_call(
        _inproj_kernel,
        grid=(N // _RB,),
        in_specs=[
            pl.BlockSpec((_RB, D_FEAT), lambda i: (i, 0)),
            pl.BlockSpec((_RB, WALK), lambda i: (i, 0)),
            pl.BlockSpec((RWSE_DIM, WALK), lambda i: (0, 0)),
            pl.BlockSpec((1, RWSE_DIM), lambda i: (0, 0)),
            pl.BlockSpec((HID, D_FEAT), lambda i: (0, 0)),
            pl.BlockSpec((HID, RWSE_DIM), lambda i: (0, 0)),
            pl.BlockSpec((1, HID), lambda i: (0, 0)),
            pl.BlockSpec((4 * HID, HID), lambda i: (0, 0)),
            pl.BlockSpec((1, 4 * HID), lambda i: (0, 0)),
        ],
        out_specs=[pl.BlockSpec((_RB, HID), lambda i: (i, 0)),
                   pl.BlockSpec((_RB, 4 * HID), lambda i: (i, 0))],
        out_shape=[jax.ShapeDtypeStruct((N, HID), jnp.float32),
                   jax.ShapeDtypeStruct((N, 4 * HID), jnp.float32)],
    )(x, diags, w_rwse, b_rwse, w1, w2, b_in, wcat, bcat)


_BA = 512


def _attn_kernel(q_ref, kv_ref, a_ref, num_ref, den_ref):
    @pl.when(pl.program_id(1) == 0)
    def _():
        num_ref[...] = jnp.zeros_like(num_ref)
        den_ref[...] = jnp.zeros_like(den_ref)

    at = a_ref[...]
    for h in range(HEADS):
        sl = slice(h * DH, (h + 1) * DH)
        ksl = slice(HID + h * DH, HID + (h + 1) * DH)
        vsl = slice(2 * HID + h * DH, 2 * HID + (h + 1) * DH)
        st = _cdot(kv_ref[:, ksl], q_ref[:, sl])
        w = at * jnp.exp(st * 0.25)
        num_ref[:, sl] += lax.dot_general(
            w, kv_ref[:, vsl], (((0,), (0,)), ((), ())),
            preferred_element_type=jnp.float32)
        den_ref[:, h:h + 1] += jnp.sum(w, axis=0)[:, None]


def _attention(qkvs, a):
    g = N // _BA
    return pl.pallas_call(
        _attn_kernel,
        grid=(g, g),
        in_specs=[
            pl.BlockSpec((_BA, 4 * HID), lambda i, j: (i, 0)),
            pl.BlockSpec((_BA, 4 * HID), lambda i, j: (j, 0)),
            pl.BlockSpec((_BA, _BA), lambda i, j: (j, i)),
        ],
        out_specs=[
            pl.BlockSpec((_BA, HID), lambda i, j: (i, 0)),
            pl.BlockSpec((_BA, HEADS), lambda i, j: (i, 0)),
        ],
        out_shape=[
            jax.ShapeDtypeStruct((N, HID), jnp.float32),
            jax.ShapeDtypeStruct((N, HEADS), jnp.float32),
        ],
    )(qkvs, qkvs, a)


def _combine_core(h_ref, s_ref, num_ref, den_ref, g_ref, b_ref):
    den = den_ref[...]
    denr = jnp.concatenate(
        [jnp.broadcast_to(den[:, h:h + 1], (_RB, DH)) for h in range(HEADS)],
        axis=1)
    conv = num_ref[...] / (denr + 1e-16) + s_ref[:, 3 * HID:4 * HID]
    z = h_ref[...] + conv
    mu = jnp.mean(z, axis=1, keepdims=True)
    zc = z - mu
    var = jnp.mean(zc * zc, axis=1, keepdims=True)
    hn = zc / jnp.sqrt(var + 1e-5) * g_ref[...] + b_ref[...]
    return jnp.maximum(hn, 0.0)


def _combine_proj_kernel(h_ref, s_ref, num_ref, den_ref, g_ref, b_ref,
                         wc_ref, bc_ref, o_ref, q_ref):
    res = _combine_core(h_ref, s_ref, num_ref, den_ref, g_ref, b_ref)
    o_ref[...] = res
    q_ref[...] = _cdot(res, wc_ref[...]) + bc_ref[...]


def _combine_pad_kernel(h_ref, s_ref, num_ref, den_ref, g_ref, b_ref, o_ref):
    res = _combine_core(h_ref, s_ref, num_ref, den_ref, g_ref, b_ref)
    o_ref[...] = jnp.concatenate([res, jnp.zeros_like(res)], axis=1)


_COMBINE_IN = [
    pl.BlockSpec((_RB, HID), lambda i: (i, 0)),
    pl.BlockSpec((_RB, 4 * HID), lambda i: (i, 0)),
    pl.BlockSpec((_RB, HID), lambda i: (i, 0)),
    pl.BlockSpec((_RB, HEADS), lambda i: (i, 0)),
    pl.BlockSpec((1, HID), lambda i: (0, 0)),
    pl.BlockSpec((1, HID), lambda i: (0, 0)),
]


def _combine_proj(h, qkvs, num, den, g, be, wcat, bcat):
    return pl.pallas_call(
        _combine_proj_kernel,
        grid=(N // _RB,),
        in_specs=_COMBINE_IN + [
            pl.BlockSpec((4 * HID, HID), lambda i: (0, 0)),
            pl.BlockSpec((1, 4 * HID), lambda i: (0, 0)),
        ],
        out_specs=[pl.BlockSpec((_RB, HID), lambda i: (i, 0)),
                   pl.BlockSpec((_RB, 4 * HID), lambda i: (i, 0))],
        out_shape=[jax.ShapeDtypeStruct((N, HID), jnp.float32),
                   jax.ShapeDtypeStruct((N, 4 * HID), jnp.float32)],
    )(h, qkvs, num, den, g, be, wcat, bcat)


def _combine_pad(h, qkvs, num, den, g, be):
    return pl.pallas_call(
        _combine_pad_kernel,
        grid=(N // _RB,),
        in_specs=_COMBINE_IN,
        out_specs=pl.BlockSpec((_RB, 2 * HID), lambda i: (i, 0)),
        out_shape=jax.ShapeDtypeStruct((N, 2 * HID), jnp.float32),
    )(h, qkvs, num, den, g, be)


PAIRS_PER = NPAIRS // 32


@functools.cache
def _pairs_kernel():
    return functools.partial(
        pl.kernel,
        out_type=jax.ShapeDtypeStruct((NPAIRS,), jnp.float32),
        mesh=_sc_mesh(),
        scratch_types=[
            pltpu.VMEM((PAIRS_PER,), jnp.int32),
            pltpu.VMEM((PAIRS_PER,), jnp.int32),
            pltpu.VMEM((PAIRS_PER, 2 * HID), jnp.float32),
            pltpu.VMEM((PAIRS_PER, 2 * HID), jnp.float32),
            pltpu.VMEM((PAIRS_PER,), jnp.float32),
            pltpu.SemaphoreType.DMA,
        ],
        compiler_params=_SC_PARAMS,
    )(_pairs_body)


def _pairs(h, src, dst):
    return _pairs_kernel()(h, src, dst)


def _pairs_body(h_hbm, src_hbm, dst_hbm, out_hbm, s_v, d_v, hs_v, hd_v, res_v,
                sem):
    wid = lax.axis_index("s") * 2 + lax.axis_index("c")
    base = wid * PAIRS_PER
    pltpu.sync_copy(src_hbm.at[pl.ds(base, PAIRS_PER)], s_v)
    pltpu.sync_copy(dst_hbm.at[pl.ds(base, PAIRS_PER)], d_v)
    pltpu.async_copy(h_hbm.at[s_v], hs_v, sem).wait()
    pltpu.async_copy(h_hbm.at[d_v], hd_v, sem).wait()

    def group_body(g, _):
        rows = g * 16 + lax.iota(jnp.int32, 16)
        acc = jnp.zeros((16,), jnp.float32)
        for c in range(HID):
            colv = jnp.full((16,), c, jnp.int32)
            acc = acc + (plsc.load_gather(hs_v, [rows, colv])
                         * plsc.load_gather(hd_v, [rows, colv]))
        res_v[pl.ds(g * 16, 16)] = 1.0 / (1.0 + jnp.exp(-acc))
        return 0

    lax.fori_loop(0, PAIRS_PER // 16, group_body, 0)
    pltpu.sync_copy(res_v, out_hbm.at[pl.ds(base, PAIRS_PER)])


def kernel(x, edge_index, src, dst, W_rwse, b_rwse, W_in, b_in,
           Wq0, bq0, Wk0, bk0, Wv0, bv0, Ws0, bs0, g0, be0,
           Wq1, bq1, Wk1, bk1, Wv1, bv1, Ws1, bs1, g1, be1):
    row = edge_index[0]
    col = edge_index[1]
    a = _build_counts(row, col)
    p = _normalize(a)
    p2 = _mm(p, p)
    p3, p4 = _mm2(p2, p, p2)
    p8 = _mm(p4, p4)
    p12 = _mm(p8, p4)
    diags = _diag_pairs(p, p2, p3, p4, p8, p12)
    wcat0 = jnp.concatenate([Wq0, Wk0, Wv0, Ws0], axis=0)
    bcat0 = jnp.concatenate([bq0, bk0, bv0, bs0]).reshape(1, -1)
    wcat1 = jnp.concatenate([Wq1, Wk1, Wv1, Ws1], axis=0)
    bcat1 = jnp.concatenate([bq1, bk1, bv1, bs1]).reshape(1, -1)
    h, qkvs0 = _input_proj(x, diags, W_rwse, b_rwse.reshape(1, -1),
                           W_in[:, :D_FEAT], W_in[:, D_FEAT:],
                           b_in.reshape(1, -1), wcat0, bcat0)
    num, den = _attention(qkvs0, a)
    h, qkvs1 = _combine_proj(h, qkvs0, num, den, g0.reshape(1, -1),
                             be0.reshape(1, -1), wcat1, bcat1)
    num, den = _attention(qkvs1, a)
    hp = _combine_pad(h, qkvs1, num, den, g1.reshape(1, -1),
                      be1.reshape(1, -1))
    return _pairs(hp, src, dst)


# speedup vs baseline: 14.1548x; 1.0080x over previous
import functools

import jax
import jax.numpy as jnp
from jax import lax
from jax.experimental import pallas as pl
from jax.experimental.pallas import tpu as pltpu
from jax.experimental.pallas import tpu_sc as plsc

N = 2048
E = 32768
D_FEAT = 128
HID = 64
HEADS = 4
DH = 16
WALK = 16
RWSE_DIM = 16
NPAIRS = 4096

_SC_PARAMS = pltpu.CompilerParams(needs_layout_passes=False)


@functools.cache
def _sc_mesh():
    return plsc.VectorSubcoreMesh(core_axis_name="c", subcore_axis_name="s")


ROWS_PER_TILE = 64
COLS_PER_PASS = 1024
ECHUNK = 8192


@functools.cache
def _build_counts_kernel():
    return functools.partial(
        pl.kernel,
        out_type=jax.ShapeDtypeStruct((N, N), jnp.float32),
        mesh=_sc_mesh(),
        scratch_types=[
            pltpu.VMEM((ROWS_PER_TILE, COLS_PER_PASS), jnp.float32),
            pltpu.VMEM((ECHUNK,), jnp.int32),
            pltpu.VMEM((ECHUNK,), jnp.int32),
        ],
        compiler_params=_SC_PARAMS,
    )(_build_counts_body)


def _build_counts(row, col):
    return _build_counts_kernel()(row, col)


def _build_counts_body(row_hbm, col_hbm, out_hbm, acc_v, r_v, c_v):
    wid = lax.axis_index("s") * 2 + lax.axis_index("c")
    r0 = wid * ROWS_PER_TILE
    zeros16 = jnp.zeros((16,), jnp.float32)
    ones16 = jnp.ones((16,), jnp.float32)
    for p in range(N // COLS_PER_PASS):
        c0 = p * COLS_PER_PASS

        def zero_body(i, _):
            r = i // 8
            cbase = (i % 8) * 128
            for u in range(8):
                acc_v[r, pl.ds(cbase + u * 16, 16)] = zeros16
            return 0

        lax.fori_loop(0, ROWS_PER_TILE * COLS_PER_PASS // (16 * 8),
                      zero_body, 0)

        def chunk_body(ch, _):
            pltpu.sync_copy(row_hbm.at[pl.ds(ch * ECHUNK, ECHUNK)], r_v)
            pltpu.sync_copy(col_hbm.at[pl.ds(ch * ECHUNK, ECHUNK)], c_v)

            def group_body(g, _):
                for u in range(4):
                    o = (g * 4 + u) * 16
                    r16 = r_v[pl.ds(o, 16)]
                    c16 = c_v[pl.ds(o, 16)]
                    m = ((r16 >= r0) & (r16 < r0 + ROWS_PER_TILE)
                         & (c16 >= c0) & (c16 < c0 + COLS_PER_PASS))
                    rr = jnp.where(m, r16 - r0, 0)
                    cc = jnp.where(m, c16 - c0, 0)
                    plsc.addupdate_scatter(acc_v, [rr, cc], ones16, mask=m)
                return 0

            lax.fori_loop(0, ECHUNK // (16 * 4), group_body, 0)
            return 0

        lax.fori_loop(0, E // ECHUNK, chunk_body, 0)
        pltpu.sync_copy(
            acc_v, out_hbm.at[pl.ds(r0, ROWS_PER_TILE), pl.ds(c0, COLS_PER_PASS)])


def _norm_kernel(a_ref, p_ref):
    a = a_ref[...]
    deg = jnp.sum(a, axis=1, keepdims=True)
    p_ref[...] = (a / jnp.maximum(deg, 1.0)).astype(jnp.bfloat16)


def _normalize(a):
    return pl.pallas_call(
        _norm_kernel,
        grid=(8,),
        in_specs=[pl.BlockSpec((N // 8, N), lambda i: (i, 0))],
        out_specs=pl.BlockSpec((N // 8, N), lambda i: (i, 0)),
        out_shape=jax.ShapeDtypeStruct((N, N), jnp.bfloat16),
    )(a)


_MB = 512


def _mm_kernel(a_ref, b_ref, o_ref, acc_ref):
    @pl.when(pl.program_id(2) == 0)
    def _():
        acc_ref[...] = jnp.zeros_like(acc_ref)

    acc_ref[...] += jnp.dot(a_ref[...], b_ref[...],
                            preferred_element_type=jnp.float32)

    @pl.when(pl.program_id(2) == pl.num_programs(2) - 1)
    def _():
        o_ref[...] = acc_ref[...].astype(jnp.bfloat16)


def _mm(a, b):
    g = N // _MB
    return pl.pallas_call(
        _mm_kernel,
        grid=(g, g, g),
        in_specs=[pl.BlockSpec((_MB, _MB), lambda i, j, k: (i, k)),
                  pl.BlockSpec((_MB, _MB), lambda i, j, k: (k, j))],
        out_specs=pl.BlockSpec((_MB, _MB), lambda i, j, k: (i, j)),
        out_shape=jax.ShapeDtypeStruct((N, N), jnp.bfloat16),
        scratch_shapes=[pltpu.VMEM((_MB, _MB), jnp.float32)],
    )(a, b)


def _mm2_kernel(a_ref, b1_ref, b2_ref, o1_ref, o2_ref, acc1_ref, acc2_ref):
    @pl.when(pl.program_id(2) == 0)
    def _():
        acc1_ref[...] = jnp.zeros_like(acc1_ref)
        acc2_ref[...] = jnp.zeros_like(acc2_ref)

    a = a_ref[...]
    acc1_ref[...] += jnp.dot(a, b1_ref[...],
                             preferred_element_type=jnp.float32)
    acc2_ref[...] += jnp.dot(a, b2_ref[...],
                             preferred_element_type=jnp.float32)

    @pl.when(pl.program_id(2) == pl.num_programs(2) - 1)
    def _():
        o1_ref[...] = acc1_ref[...].astype(jnp.bfloat16)
        o2_ref[...] = acc2_ref[...].astype(jnp.bfloat16)


def _mm2(a, b1, b2):
    g = N // _MB
    sd = jax.ShapeDtypeStruct((N, N), jnp.bfloat16)
    return pl.pallas_call(
        _mm2_kernel,
        grid=(g, g, g),
        in_specs=[pl.BlockSpec((_MB, _MB), lambda i, j, k: (i, k)),
                  pl.BlockSpec((_MB, _MB), lambda i, j, k: (k, j)),
                  pl.BlockSpec((_MB, _MB), lambda i, j, k: (k, j))],
        out_specs=[pl.BlockSpec((_MB, _MB), lambda i, j, k: (i, j)),
                   pl.BlockSpec((_MB, _MB), lambda i, j, k: (i, j))],
        out_shape=[sd, sd],
        scratch_shapes=[pltpu.VMEM((_MB, _MB), jnp.float32),
                        pltpu.VMEM((_MB, _MB), jnp.float32)],
    )(a, b1, b2)


_BT = 128
_DK = 512


def _diag_kernel(p1_ref, p2_ref, p3_ref, p4r_ref, p4c_ref, p8c_ref, p12c_ref,
                 o_ref, acc_ref):
    bi = pl.program_id(0)
    kk = pl.program_id(1)
    nk = pl.num_programs(1)

    @pl.when(kk == 0)
    def _():
        o_ref[...] = jnp.zeros_like(o_ref)
        acc_ref[...] = jnp.zeros_like(acc_ref)

    a_chunks = [p1_ref[...], p2_ref[...], p3_ref[...], p4r_ref[...]]
    b4 = p4c_ref[...]
    b8 = p8c_ref[...]
    b12 = p12c_ref[...]

    def dot(a, b):
        return jnp.dot(a, b, preferred_element_type=jnp.float32)

    for t, a in enumerate(a_chunks[:3]):
        acc_ref[t * _BT:(t + 1) * _BT, :] += dot(a, b4)
        acc_ref[(3 + t) * _BT:(4 + t) * _BT, :] += dot(a, b8)
    for t, a in enumerate(a_chunks):
        acc_ref[(6 + t) * _BT:(7 + t) * _BT, :] += dot(a, b12)

    ii = lax.broadcasted_iota(jnp.int32, (_BT, _BT), 0)
    jj = lax.broadcasted_iota(jnp.int32, (_BT, _BT), 1)
    eye = (ii == jj).astype(jnp.float32)

    @pl.when(kk == bi // 4)
    def _():
        off = (bi % 4) * _BT
        iw = lax.broadcasted_iota(jnp.int32, (_BT, _DK), 0)
        jw = lax.broadcasted_iota(jnp.int32, (_BT, _DK), 1)
        mask_a = (jw == iw + off).astype(jnp.float32)
        it = lax.broadcasted_iota(jnp.int32, (_DK, _BT), 0)
        jt = lax.broadcasted_iota(jnp.int32, (_DK, _BT), 1)
        mask_b = (it == jt + off).astype(jnp.float32)
        z = jnp.zeros((_BT, 1), jnp.float32)

        def rs_a(x):
            return jnp.sum(x.astype(jnp.float32) * mask_a, axis=1,
                           keepdims=True)

        def rs_b(x):
            return jnp.sum(x.astype(jnp.float32) * mask_b, axis=0)[:, None]

        o_ref[...] += jnp.concatenate(
            [rs_a(a_chunks[0]), rs_a(a_chunks[1]), rs_a(a_chunks[2]),
             rs_a(a_chunks[3]), z, z, z, rs_b(b8), z, z, z, rs_b(b12),
             z, z, z, z], axis=1)

    @pl.when(kk == nk - 1)
    def _():
        acc = acc_ref[...]

        def dg(t):
            sub = acc[t * _BT:(t + 1) * _BT, :]
            return jnp.sum(sub * eye, axis=1, keepdims=True)

        z = jnp.zeros((_BT, 1), jnp.float32)
        o_ref[...] += jnp.concatenate(
            [z, z, z, z, dg(0), dg(1), dg(2), z, dg(3), dg(4), dg(5), z,
             dg(6), dg(7), dg(8), dg(9)], axis=1)


def _diag_pairs(p1, p2, p3, p4, p8, p12):
    row = pl.BlockSpec((_BT, _DK), lambda i, k: (i, k))
    col = pl.BlockSpec((_DK, _BT), lambda i, k: (k, i))
    return pl.pallas_call(
        _diag_kernel,
        grid=(N // _BT, N // _DK),
        in_specs=[row, row, row, row, col, col, col],
        out_specs=pl.BlockSpec((_BT, WALK), lambda i, k: (i, 0)),
        out_shape=jax.ShapeDtypeStruct((N, WALK), jnp.float32),
        scratch_shapes=[pltpu.VMEM((10 * _BT, _BT), jnp.float32)],
    )(p1, p2, p3, p4, p4, p8, p12)


_RB = 256


def _cdot(a, b):
    return lax.dot_general(a, b, (((1,), (1,)), ((), ())),
                           preferred_element_type=jnp.float32)


def _inproj_kernel(x_ref, dg_ref, wr_ref, br_ref, w1_ref, w2_ref, b_ref,
                   wc_ref, bc_ref, h_ref, q_ref):
    pe = _cdot(dg_ref[...], wr_ref[...]) + br_ref[...]
    h = _cdot(x_ref[...], w1_ref[...]) + _cdot(pe, w2_ref[...]) + b_ref[...]
    h_ref[...] = h
    q_ref[...] = _cdot(h, wc_ref[...]) + bc_ref[...]


def _input_proj(x, diags, w_rwse, b_rwse, w1, w2, b_in, wcat, bcat):
    return pl.pallas_call(
        _inproj_kernel,
        grid=(N // _RB,),
        in_specs=[
            pl.BlockSpec((_RB, D_FEAT), lambda i: (i, 0)),
            pl.BlockSpec((_RB, WALK), lambda i: (i, 0)),
            pl.BlockSpec((RWSE_DIM, WALK), lambda i: (0, 0)),
            pl.BlockSpec((1, RWSE_DIM), lambda i: (0, 0)),
            pl.BlockSpec((HID, D_FEAT), lambda i: (0, 0)),
            pl.BlockSpec((HID, RWSE_DIM), lambda i: (0, 0)),
            pl.BlockSpec((1, HID), lambda i: (0, 0)),
            pl.BlockSpec((4 * HID, HID), lambda i: (0, 0)),
            pl.BlockSpec((1, 4 * HID), lambda i: (0, 0)),
        ],
        out_specs=[pl.BlockSpec((_RB, HID), lambda i: (i, 0)),
                   pl.BlockSpec((_RB, 4 * HID), lambda i: (i, 0))],
        out_shape=[jax.ShapeDtypeStruct((N, HID), jnp.float32),
                   jax.ShapeDtypeStruct((N, 4 * HID), jnp.float32)],
    )(x, diags, w_rwse, b_rwse, w1, w2, b_in, wcat, bcat)


_BA = 512


def _attn_accum(q_ref, kv_ref, a_ref, num_ref, den_ref):
    @pl.when(pl.program_id(1) == 0)
    def _():
        num_ref[...] = jnp.zeros_like(num_ref)
        den_ref[...] = jnp.zeros_like(den_ref)

    at = a_ref[...]
    for h in range(HEADS):
        sl = slice(h * DH, (h + 1) * DH)
        ksl = slice(HID + h * DH, HID + (h + 1) * DH)
        vsl = slice(2 * HID + h * DH, 2 * HID + (h + 1) * DH)
        st = _cdot(kv_ref[:, ksl], q_ref[:, sl])
        w = at * jnp.exp(st * 0.25)
        num_ref[:, sl] += lax.dot_general(
            w, kv_ref[:, vsl], (((0,), (0,)), ((), ())),
            preferred_element_type=jnp.float32)
        den_ref[:, h:h + 1] += jnp.sum(w, axis=0)[:, None]


def _conv_core(h_ref, s_ref, num, den, g_ref, b_ref, rows):
    denr = jnp.concatenate(
        [jnp.broadcast_to(den[:, h:h + 1], (rows, DH)) for h in range(HEADS)],
        axis=1)
    conv = num / (denr + 1e-16) + s_ref[:, 3 * HID:4 * HID]
    z = h_ref[...] + conv
    mu = jnp.mean(z, axis=1, keepdims=True)
    zc = z - mu
    var = jnp.mean(zc * zc, axis=1, keepdims=True)
    hn = zc / jnp.sqrt(var + 1e-5) * g_ref[...] + b_ref[...]
    return jnp.maximum(hn, 0.0)


def _attn_proj_kernel(q_ref, kv_ref, a_ref, h_ref, g_ref, b_ref, wc_ref,
                      bc_ref, o_ref, qo_ref, num_ref, den_ref):
    _attn_accum(q_ref, kv_ref, a_ref, num_ref, den_ref)

    @pl.when(pl.program_id(1) == pl.num_programs(1) - 1)
    def _():
        res = _conv_core(h_ref, q_ref, num_ref[...], den_ref[...], g_ref,
                         b_ref, _BA)
        o_ref[...] = res
        qo_ref[...] = _cdot(res, wc_ref[...]) + bc_ref[...]


def _attn_pad_kernel(q_ref, kv_ref, a_ref, h_ref, g_ref, b_ref, o_ref,
                     num_ref, den_ref):
    _attn_accum(q_ref, kv_ref, a_ref, num_ref, den_ref)

    @pl.when(pl.program_id(1) == pl.num_programs(1) - 1)
    def _():
        res = _conv_core(h_ref, q_ref, num_ref[...], den_ref[...], g_ref,
                         b_ref, _BA)
        o_ref[...] = jnp.concatenate([res, jnp.zeros_like(res)], axis=1)


_ATTN_IN = [
    pl.BlockSpec((_BA, 4 * HID), lambda i, j: (i, 0)),
    pl.BlockSpec((_BA, 4 * HID), lambda i, j: (j, 0)),
    pl.BlockSpec((_BA, _BA), lambda i, j: (j, i)),
    pl.BlockSpec((_BA, HID), lambda i, j: (i, 0)),
    pl.BlockSpec((1, HID), lambda i, j: (0, 0)),
    pl.BlockSpec((1, HID), lambda i, j: (0, 0)),
]
_ATTN_SCRATCH = [pltpu.VMEM((_BA, HID), jnp.float32),
                 pltpu.VMEM((_BA, HEADS), jnp.float32)]


def _attn_proj(qkvs, a, h, g, be, wcat, bcat):
    gr = N // _BA
    return pl.pallas_call(
        _attn_proj_kernel,
        grid=(gr, gr),
        in_specs=_ATTN_IN + [
            pl.BlockSpec((4 * HID, HID), lambda i, j: (0, 0)),
            pl.BlockSpec((1, 4 * HID), lambda i, j: (0, 0)),
        ],
        out_specs=[pl.BlockSpec((_BA, HID), lambda i, j: (i, 0)),
                   pl.BlockSpec((_BA, 4 * HID), lambda i, j: (i, 0))],
        out_shape=[jax.ShapeDtypeStruct((N, HID), jnp.float32),
                   jax.ShapeDtypeStruct((N, 4 * HID), jnp.float32)],
        scratch_shapes=_ATTN_SCRATCH,
    )(qkvs, qkvs, a, h, g, be, wcat, bcat)


def _attn_pad(qkvs, a, h, g, be):
    gr = N // _BA
    return pl.pallas_call(
        _attn_pad_kernel,
        grid=(gr, gr),
        in_specs=_ATTN_IN,
        out_specs=pl.BlockSpec((_BA, 2 * HID), lambda i, j: (i, 0)),
        out_shape=jax.ShapeDtypeStruct((N, 2 * HID), jnp.float32),
        scratch_shapes=_ATTN_SCRATCH,
    )(qkvs, qkvs, a, h, g, be)


PAIRS_PER = NPAIRS // 32


@functools.cache
def _pairs_kernel():
    return functools.partial(
        pl.kernel,
        out_type=jax.ShapeDtypeStruct((NPAIRS,), jnp.float32),
        mesh=_sc_mesh(),
        scratch_types=[
            pltpu.VMEM((PAIRS_PER,), jnp.int32),
            pltpu.VMEM((PAIRS_PER,), jnp.int32),
            pltpu.VMEM((PAIRS_PER, 2 * HID), jnp.float32),
            pltpu.VMEM((PAIRS_PER, 2 * HID), jnp.float32),
            pltpu.VMEM((PAIRS_PER,), jnp.float32),
            pltpu.SemaphoreType.DMA,
        ],
        compiler_params=_SC_PARAMS,
    )(_pairs_body)


def _pairs(h, src, dst):
    return _pairs_kernel()(h, src, dst)


def _pairs_body(h_hbm, src_hbm, dst_hbm, out_hbm, s_v, d_v, hs_v, hd_v, res_v,
                sem):
    wid = lax.axis_index("s") * 2 + lax.axis_index("c")
    base = wid * PAIRS_PER
    pltpu.sync_copy(src_hbm.at[pl.ds(base, PAIRS_PER)], s_v)
    pltpu.sync_copy(dst_hbm.at[pl.ds(base, PAIRS_PER)], d_v)
    pltpu.async_copy(h_hbm.at[s_v], hs_v, sem).wait()
    pltpu.async_copy(h_hbm.at[d_v], hd_v, sem).wait()

    def group_body(g, _):
        rows = g * 16 + lax.iota(jnp.int32, 16)
        acc = jnp.zeros((16,), jnp.float32)
        for c in range(HID):
            colv = jnp.full((16,), c, jnp.int32)
            acc = acc + (plsc.load_gather(hs_v, [rows, colv])
                         * plsc.load_gather(hd_v, [rows, colv]))
        res_v[pl.ds(g * 16, 16)] = 1.0 / (1.0 + jnp.exp(-acc))
        return 0

    lax.fori_loop(0, PAIRS_PER // 16, group_body, 0)
    pltpu.sync_copy(res_v, out_hbm.at[pl.ds(base, PAIRS_PER)])


def kernel(x, edge_index, src, dst, W_rwse, b_rwse, W_in, b_in,
           Wq0, bq0, Wk0, bk0, Wv0, bv0, Ws0, bs0, g0, be0,
           Wq1, bq1, Wk1, bk1, Wv1, bv1, Ws1, bs1, g1, be1):
    row = edge_index[0]
    col = edge_index[1]
    a = _build_counts(row, col)
    p = _normalize(a)
    p2 = _mm(p, p)
    p3, p4 = _mm2(p2, p, p2)
    p8 = _mm(p4, p4)
    p12 = _mm(p8, p4)
    diags = _diag_pairs(p, p2, p3, p4, p8, p12)
    wcat0 = jnp.concatenate([Wq0, Wk0, Wv0, Ws0], axis=0)
    bcat0 = jnp.concatenate([bq0, bk0, bv0, bs0]).reshape(1, -1)
    wcat1 = jnp.concatenate([Wq1, Wk1, Wv1, Ws1], axis=0)
    bcat1 = jnp.concatenate([bq1, bk1, bv1, bs1]).reshape(1, -1)
    h, qkvs0 = _input_proj(x, diags, W_rwse, b_rwse.reshape(1, -1),
                           W_in[:, :D_FEAT], W_in[:, D_FEAT:],
                           b_in.reshape(1, -1), wcat0, bcat0)
    h, qkvs1 = _attn_proj(qkvs0, a, h, g0.reshape(1, -1),
                          be0.reshape(1, -1), wcat1, bcat1)
    hp = _attn_pad(qkvs1, a, h, g1.reshape(1, -1), be1.reshape(1, -1))
    return _pairs(hp, src, dst)


# speedup vs baseline: 14.6061x; 1.0319x over previous
import functools

import jax
import jax.numpy as jnp
from jax import lax
from jax.experimental import pallas as pl
from jax.experimental.pallas import tpu as pltpu
from jax.experimental.pallas import tpu_sc as plsc

N = 2048
E = 32768
D_FEAT = 128
HID = 64
HEADS = 4
DH = 16
WALK = 16
RWSE_DIM = 16
NPAIRS = 4096

_SC_PARAMS = pltpu.CompilerParams(needs_layout_passes=False)


@functools.cache
def _sc_mesh():
    return plsc.VectorSubcoreMesh(core_axis_name="c", subcore_axis_name="s")


ROWS_PER_TILE = 64
COLS_PER_PASS = 1024
ECHUNK = 8192


@functools.cache
def _build_counts_kernel():
    return functools.partial(
        pl.kernel,
        out_type=jax.ShapeDtypeStruct((N, N), jnp.float32),
        mesh=_sc_mesh(),
        scratch_types=[
            pltpu.VMEM((ROWS_PER_TILE, COLS_PER_PASS), jnp.float32),
            pltpu.VMEM((ECHUNK,), jnp.int32),
            pltpu.VMEM((ECHUNK,), jnp.int32),
            pltpu.VMEM((ECHUNK,), jnp.int32),
            pltpu.VMEM((ECHUNK,), jnp.int32),
            pltpu.SemaphoreType.DMA,
            pltpu.SemaphoreType.DMA,
        ],
        compiler_params=_SC_PARAMS,
    )(_build_counts_body)


def _build_counts(row, col):
    return _build_counts_kernel()(row, col)


def _build_counts_body(row_hbm, col_hbm, out_hbm, acc_v, r_a, c_a, r_b, c_b,
                       sem_a, sem_b):
    wid = lax.axis_index("s") * 2 + lax.axis_index("c")
    r0 = wid * ROWS_PER_TILE
    zeros16 = jnp.zeros((16,), jnp.float32)
    ones16 = jnp.ones((16,), jnp.float32)
    nch = E // ECHUNK
    bufs = ((r_a, c_a, sem_a), (r_b, c_b, sem_b))

    def stage(ch):
        rv, cv, sem = bufs[ch % 2]
        h1 = pltpu.async_copy(row_hbm.at[pl.ds(ch * ECHUNK, ECHUNK)], rv, sem)
        h2 = pltpu.async_copy(col_hbm.at[pl.ds(ch * ECHUNK, ECHUNK)], cv, sem)
        return h1, h2

    for p in range(N // COLS_PER_PASS):
        c0 = p * COLS_PER_PASS
        pend = stage(0)

        def zero_body(i, _):
            r = i // 8
            cbase = (i % 8) * 128
            for u in range(8):
                acc_v[r, pl.ds(cbase + u * 16, 16)] = zeros16
            return 0

        lax.fori_loop(0, ROWS_PER_TILE * COLS_PER_PASS // (16 * 8),
                      zero_body, 0)

        for ch in range(nch):
            rv, cv, _ = bufs[ch % 2]
            cur = pend
            if ch + 1 < nch:
                pend = stage(ch + 1)
            cur[0].wait()
            cur[1].wait()

            def group_body(g, _):
                for u in range(4):
                    o = (g * 4 + u) * 16
                    r16 = rv[pl.ds(o, 16)]
                    c16 = cv[pl.ds(o, 16)]
                    m = ((r16 >= r0) & (r16 < r0 + ROWS_PER_TILE)
                         & (c16 >= c0) & (c16 < c0 + COLS_PER_PASS))
                    rr = jnp.where(m, r16 - r0, 0)
                    cc = jnp.where(m, c16 - c0, 0)
                    plsc.addupdate_scatter(acc_v, [rr, cc], ones16, mask=m)
                return 0

            lax.fori_loop(0, ECHUNK // (16 * 4), group_body, 0)

        pltpu.sync_copy(
            acc_v, out_hbm.at[pl.ds(r0, ROWS_PER_TILE), pl.ds(c0, COLS_PER_PASS)])


def _norm_kernel(a_ref, p_ref):
    a = a_ref[...]
    deg = jnp.sum(a, axis=1, keepdims=True)
    p_ref[...] = (a / jnp.maximum(deg, 1.0)).astype(jnp.bfloat16)


def _normalize(a):
    return pl.pallas_call(
        _norm_kernel,
        grid=(8,),
        in_specs=[pl.BlockSpec((N // 8, N), lambda i: (i, 0))],
        out_specs=pl.BlockSpec((N // 8, N), lambda i: (i, 0)),
        out_shape=jax.ShapeDtypeStruct((N, N), jnp.bfloat16),
    )(a)


_MB = 512


def _mm_kernel(a_ref, b_ref, o_ref, acc_ref):
    @pl.when(pl.program_id(2) == 0)
    def _():
        acc_ref[...] = jnp.zeros_like(acc_ref)

    acc_ref[...] += jnp.dot(a_ref[...], b_ref[...],
                            preferred_element_type=jnp.float32)

    @pl.when(pl.program_id(2) == pl.num_programs(2) - 1)
    def _():
        o_ref[...] = acc_ref[...].astype(jnp.bfloat16)


def _mm(a, b):
    g = N // _MB
    return pl.pallas_call(
        _mm_kernel,
        grid=(g, g, g),
        in_specs=[pl.BlockSpec((_MB, _MB), lambda i, j, k: (i, k)),
                  pl.BlockSpec((_MB, _MB), lambda i, j, k: (k, j))],
        out_specs=pl.BlockSpec((_MB, _MB), lambda i, j, k: (i, j)),
        out_shape=jax.ShapeDtypeStruct((N, N), jnp.bfloat16),
        scratch_shapes=[pltpu.VMEM((_MB, _MB), jnp.float32)],
    )(a, b)


def _mm2_kernel(a_ref, b1_ref, b2_ref, o1_ref, o2_ref, acc1_ref, acc2_ref):
    @pl.when(pl.program_id(2) == 0)
    def _():
        acc1_ref[...] = jnp.zeros_like(acc1_ref)
        acc2_ref[...] = jnp.zeros_like(acc2_ref)

    a = a_ref[...]
    acc1_ref[...] += jnp.dot(a, b1_ref[...],
                             preferred_element_type=jnp.float32)
    acc2_ref[...] += jnp.dot(a, b2_ref[...],
                             preferred_element_type=jnp.float32)

    @pl.when(pl.program_id(2) == pl.num_programs(2) - 1)
    def _():
        o1_ref[...] = acc1_ref[...].astype(jnp.bfloat16)
        o2_ref[...] = acc2_ref[...].astype(jnp.bfloat16)


def _mm2(a, b1, b2):
    g = N // _MB
    sd = jax.ShapeDtypeStruct((N, N), jnp.bfloat16)
    return pl.pallas_call(
        _mm2_kernel,
        grid=(g, g, g),
        in_specs=[pl.BlockSpec((_MB, _MB), lambda i, j, k: (i, k)),
                  pl.BlockSpec((_MB, _MB), lambda i, j, k: (k, j)),
                  pl.BlockSpec((_MB, _MB), lambda i, j, k: (k, j))],
        out_specs=[pl.BlockSpec((_MB, _MB), lambda i, j, k: (i, j)),
                   pl.BlockSpec((_MB, _MB), lambda i, j, k: (i, j))],
        out_shape=[sd, sd],
        scratch_shapes=[pltpu.VMEM((_MB, _MB), jnp.float32),
                        pltpu.VMEM((_MB, _MB), jnp.float32)],
    )(a, b1, b2)


_BT = 128
_DK = 512


def _diag_kernel(p1_ref, p2_ref, p3_ref, p4r_ref, p4c_ref, p8c_ref, p12c_ref,
                 o_ref, acc_ref):
    bi = pl.program_id(0)
    kk = pl.program_id(1)
    nk = pl.num_programs(1)

    @pl.when(kk == 0)
    def _():
        o_ref[...] = jnp.zeros_like(o_ref)
        acc_ref[...] = jnp.zeros_like(acc_ref)

    a_chunks = [p1_ref[...], p2_ref[...], p3_ref[...], p4r_ref[...]]
    b4 = p4c_ref[...]
    b8 = p8c_ref[...]
    b12 = p12c_ref[...]

    def dot(a, b):
        return jnp.dot(a, b, preferred_element_type=jnp.float32)

    for t, a in enumerate(a_chunks[:3]):
        acc_ref[t * _BT:(t + 1) * _BT, :] += dot(a, b4)
        acc_ref[(3 + t) * _BT:(4 + t) * _BT, :] += dot(a, b8)
    for t, a in enumerate(a_chunks):
        acc_ref[(6 + t) * _BT:(7 + t) * _BT, :] += dot(a, b12)

    ii = lax.broadcasted_iota(jnp.int32, (_BT, _BT), 0)
    jj = lax.broadcasted_iota(jnp.int32, (_BT, _BT), 1)
    eye = (ii == jj).astype(jnp.float32)

    @pl.when(kk == bi // 4)
    def _():
        off = (bi % 4) * _BT
        iw = lax.broadcasted_iota(jnp.int32, (_BT, _DK), 0)
        jw = lax.broadcasted_iota(jnp.int32, (_BT, _DK), 1)
        mask_a = (jw == iw + off).astype(jnp.float32)
        it = lax.broadcasted_iota(jnp.int32, (_DK, _BT), 0)
        jt = lax.broadcasted_iota(jnp.int32, (_DK, _BT), 1)
        mask_b = (it == jt + off).astype(jnp.float32)
        z = jnp.zeros((_BT, 1), jnp.float32)

        def rs_a(x):
            return jnp.sum(x.astype(jnp.float32) * mask_a, axis=1,
                           keepdims=True)

        def rs_b(x):
            return jnp.sum(x.astype(jnp.float32) * mask_b, axis=0)[:, None]

        o_ref[...] += jnp.concatenate(
            [rs_a(a_chunks[0]), rs_a(a_chunks[1]), rs_a(a_chunks[2]),
             rs_a(a_chunks[3]), z, z, z, rs_b(b8), z, z, z, rs_b(b12),
             z, z, z, z], axis=1)

    @pl.when(kk == nk - 1)
    def _():
        acc = acc_ref[...]

        def dg(t):
            sub = acc[t * _BT:(t + 1) * _BT, :]
            return jnp.sum(sub * eye, axis=1, keepdims=True)

        z = jnp.zeros((_BT, 1), jnp.float32)
        o_ref[...] += jnp.concatenate(
            [z, z, z, z, dg(0), dg(1), dg(2), z, dg(3), dg(4), dg(5), z,
             dg(6), dg(7), dg(8), dg(9)], axis=1)


def _diag_pairs(p1, p2, p3, p4, p8, p12):
    row = pl.BlockSpec((_BT, _DK), lambda i, k: (i, k))
    col = pl.BlockSpec((_DK, _BT), lambda i, k: (k, i))
    return pl.pallas_call(
        _diag_kernel,
        grid=(N // _BT, N // _DK),
        in_specs=[row, row, row, row, col, col, col],
        out_specs=pl.BlockSpec((_BT, WALK), lambda i, k: (i, 0)),
        out_shape=jax.ShapeDtypeStruct((N, WALK), jnp.float32),
        scratch_shapes=[pltpu.VMEM((10 * _BT, _BT), jnp.float32)],
    )(p1, p2, p3, p4, p4, p8, p12)


_RB = 256


def _cdot(a, b):
    return lax.dot_general(a, b, (((1,), (1,)), ((), ())),
                           preferred_element_type=jnp.float32)


def _inproj_kernel(x_ref, dg_ref, wr_ref, br_ref, w1_ref, w2_ref, b_ref,
                   wc_ref, bc_ref, h_ref, q_ref):
    pe = _cdot(dg_ref[...], wr_ref[...]) + br_ref[...]
    h = _cdot(x_ref[...], w1_ref[...]) + _cdot(pe, w2_ref[...]) + b_ref[...]
    h_ref[...] = h
    q_ref[...] = _cdot(h, wc_ref[...]) + bc_ref[...]


def _input_proj(x, diags, w_rwse, b_rwse, w1, w2, b_in, wcat, bcat):
    return pl.pallas_call(
        _inproj_kernel,
        grid=(N // _RB,),
        in_specs=[
            pl.BlockSpec((_RB, D_FEAT), lambda i: (i, 0)),
            pl.BlockSpec((_RB, WALK), lambda i: (i, 0)),
            pl.BlockSpec((RWSE_DIM, WALK), lambda i: (0, 0)),
            pl.BlockSpec((1, RWSE_DIM), lambda i: (0, 0)),
            pl.BlockSpec((HID, D_FEAT), lambda i: (0, 0)),
            pl.BlockSpec((HID, RWSE_DIM), lambda i: (0, 0)),
            pl.BlockSpec((1, HID), lambda i: (0, 0)),
            pl.BlockSpec((4 * HID, HID), lambda i: (0, 0)),
            pl.BlockSpec((1, 4 * HID), lambda i: (0, 0)),
        ],
        out_specs=[pl.BlockSpec((_RB, HID), lambda i: (i, 0)),
                   pl.BlockSpec((_RB, 4 * HID), lambda i: (i, 0))],
        out_shape=[jax.ShapeDtypeStruct((N, HID), jnp.float32),
                   jax.ShapeDtypeStruct((N, 4 * HID), jnp.float32)],
    )(x, diags, w_rwse, b_rwse, w1, w2, b_in, wcat, bcat)


_BA = 512


def _attn_accum(q_ref, kv_ref, a_ref, num_ref, den_ref):
    @pl.when(pl.program_id(1) == 0)
    def _():
        num_ref[...] = jnp.zeros_like(num_ref)
        den_ref[...] = jnp.zeros_like(den_ref)

    at = a_ref[...]
    for h in range(HEADS):
        sl = slice(h * DH, (h + 1) * DH)
        ksl = slice(HID + h * DH, HID + (h + 1) * DH)
        vsl = slice(2 * HID + h * DH, 2 * HID + (h + 1) * DH)
        st = _cdot(kv_ref[:, ksl], q_ref[:, sl])
        w = at * jnp.exp(st * 0.25)
        num_ref[:, sl] += lax.dot_general(
            w, kv_ref[:, vsl], (((0,), (0,)), ((), ())),
            preferred_element_type=jnp.float32)
        den_ref[:, h:h + 1] += jnp.sum(w, axis=0)[:, None]


def _conv_core(h_ref, s_ref, num, den, g_ref, b_ref, rows):
    denr = jnp.concatenate(
        [jnp.broadcast_to(den[:, h:h + 1], (rows, DH)) for h in range(HEADS)],
        axis=1)
    conv = num / (denr + 1e-16) + s_ref[:, 3 * HID:4 * HID]
    z = h_ref[...] + conv
    mu = jnp.mean(z, axis=1, keepdims=True)
    zc = z - mu
    var = jnp.mean(zc * zc, axis=1, keepdims=True)
    hn = zc / jnp.sqrt(var + 1e-5) * g_ref[...] + b_ref[...]
    return jnp.maximum(hn, 0.0)


def _attn_proj_kernel(q_ref, kv_ref, a_ref, h_ref, g_ref, b_ref, wc_ref,
                      bc_ref, o_ref, qo_ref, num_ref, den_ref):
    _attn_accum(q_ref, kv_ref, a_ref, num_ref, den_ref)

    @pl.when(pl.program_id(1) == pl.num_programs(1) - 1)
    def _():
        res = _conv_core(h_ref, q_ref, num_ref[...], den_ref[...], g_ref,
                         b_ref, _BA)
        o_ref[...] = res
        qo_ref[...] = _cdot(res, wc_ref[...]) + bc_ref[...]


def _attn_pad_kernel(q_ref, kv_ref, a_ref, h_ref, g_ref, b_ref, o_ref,
                     num_ref, den_ref):
    _attn_accum(q_ref, kv_ref, a_ref, num_ref, den_ref)

    @pl.when(pl.program_id(1) == pl.num_programs(1) - 1)
    def _():
        res = _conv_core(h_ref, q_ref, num_ref[...], den_ref[...], g_ref,
                         b_ref, _BA)
        o_ref[...] = jnp.concatenate([res, jnp.zeros_like(res)], axis=1)


_ATTN_IN = [
    pl.BlockSpec((_BA, 4 * HID), lambda i, j: (i, 0)),
    pl.BlockSpec((_BA, 4 * HID), lambda i, j: (j, 0)),
    pl.BlockSpec((_BA, _BA), lambda i, j: (j, i)),
    pl.BlockSpec((_BA, HID), lambda i, j: (i, 0)),
    pl.BlockSpec((1, HID), lambda i, j: (0, 0)),
    pl.BlockSpec((1, HID), lambda i, j: (0, 0)),
]
_ATTN_SCRATCH = [pltpu.VMEM((_BA, HID), jnp.float32),
                 pltpu.VMEM((_BA, HEADS), jnp.float32)]


def _attn_proj(qkvs, a, h, g, be, wcat, bcat):
    gr = N // _BA
    return pl.pallas_call(
        _attn_proj_kernel,
        grid=(gr, gr),
        in_specs=_ATTN_IN + [
            pl.BlockSpec((4 * HID, HID), lambda i, j: (0, 0)),
            pl.BlockSpec((1, 4 * HID), lambda i, j: (0, 0)),
        ],
        out_specs=[pl.BlockSpec((_BA, HID), lambda i, j: (i, 0)),
                   pl.BlockSpec((_BA, 4 * HID), lambda i, j: (i, 0))],
        out_shape=[jax.ShapeDtypeStruct((N, HID), jnp.float32),
                   jax.ShapeDtypeStruct((N, 4 * HID), jnp.float32)],
        scratch_shapes=_ATTN_SCRATCH,
    )(qkvs, qkvs, a, h, g, be, wcat, bcat)


def _attn_pad(qkvs, a, h, g, be):
    gr = N // _BA
    return pl.pallas_call(
        _attn_pad_kernel,
        grid=(gr, gr),
        in_specs=_ATTN_IN,
        out_specs=pl.BlockSpec((_BA, 2 * HID), lambda i, j: (i, 0)),
        out_shape=jax.ShapeDtypeStruct((N, 2 * HID), jnp.float32),
        scratch_shapes=_ATTN_SCRATCH,
    )(qkvs, qkvs, a, h, g, be)


PAIRS_PER = NPAIRS // 32


@functools.cache
def _pairs_kernel():
    return functools.partial(
        pl.kernel,
        out_type=jax.ShapeDtypeStruct((NPAIRS,), jnp.float32),
        mesh=_sc_mesh(),
        scratch_types=[
            pltpu.VMEM((PAIRS_PER,), jnp.int32),
            pltpu.VMEM((PAIRS_PER,), jnp.int32),
            pltpu.VMEM((PAIRS_PER, 2 * HID), jnp.float32),
            pltpu.VMEM((PAIRS_PER, 2 * HID), jnp.float32),
            pltpu.VMEM((PAIRS_PER,), jnp.float32),
            pltpu.SemaphoreType.DMA,
        ],
        compiler_params=_SC_PARAMS,
    )(_pairs_body)


def _pairs(h, src, dst):
    return _pairs_kernel()(h, src, dst)


def _pairs_body(h_hbm, src_hbm, dst_hbm, out_hbm, s_v, d_v, hs_v, hd_v, res_v,
                sem):
    wid = lax.axis_index("s") * 2 + lax.axis_index("c")
    base = wid * PAIRS_PER
    pltpu.sync_copy(src_hbm.at[pl.ds(base, PAIRS_PER)], s_v)
    pltpu.sync_copy(dst_hbm.at[pl.ds(base, PAIRS_PER)], d_v)
    pltpu.async_copy(h_hbm.at[s_v], hs_v, sem).wait()
    pltpu.async_copy(h_hbm.at[d_v], hd_v, sem).wait()

    def group_body(g, _):
        rows = g * 16 + lax.iota(jnp.int32, 16)
        acc = jnp.zeros((16,), jnp.float32)
        for c in range(HID):
            colv = jnp.full((16,), c, jnp.int32)
            acc = acc + (plsc.load_gather(hs_v, [rows, colv])
                         * plsc.load_gather(hd_v, [rows, colv]))
        res_v[pl.ds(g * 16, 16)] = 1.0 / (1.0 + jnp.exp(-acc))
        return 0

    lax.fori_loop(0, PAIRS_PER // 16, group_body, 0)
    pltpu.sync_copy(res_v, out_hbm.at[pl.ds(base, PAIRS_PER)])


def kernel(x, edge_index, src, dst, W_rwse, b_rwse, W_in, b_in,
           Wq0, bq0, Wk0, bk0, Wv0, bv0, Ws0, bs0, g0, be0,
           Wq1, bq1, Wk1, bk1, Wv1, bv1, Ws1, bs1, g1, be1):
    row = edge_index[0]
    col = edge_index[1]
    a = _build_counts(row, col)
    p = _normalize(a)
    p2 = _mm(p, p)
    p3, p4 = _mm2(p2, p, p2)
    p8 = _mm(p4, p4)
    p12 = _mm(p8, p4)
    diags = _diag_pairs(p, p2, p3, p4, p8, p12)
    wcat0 = jnp.concatenate([Wq0, Wk0, Wv0, Ws0], axis=0)
    bcat0 = jnp.concatenate([bq0, bk0, bv0, bs0]).reshape(1, -1)
    wcat1 = jnp.concatenate([Wq1, Wk1, Wv1, Ws1], axis=0)
    bcat1 = jnp.concatenate([bq1, bk1, bv1, bs1]).reshape(1, -1)
    h, qkvs0 = _input_proj(x, diags, W_rwse, b_rwse.reshape(1, -1),
                           W_in[:, :D_FEAT], W_in[:, D_FEAT:],
                           b_in.reshape(1, -1), wcat0, bcat0)
    h, qkvs1 = _attn_proj(qkvs0, a, h, g0.reshape(1, -1),
                          be0.reshape(1, -1), wcat1, bcat1)
    hp = _attn_pad(qkvs1, a, h, g1.reshape(1, -1), be1.reshape(1, -1))
    return _pairs(hp, src, dst)


# speedup vs baseline: 20.3496x; 1.3932x over previous
import functools

import jax
import jax.numpy as jnp
from jax import lax
from jax.experimental import pallas as pl
from jax.experimental.pallas import tpu as pltpu
from jax.experimental.pallas import tpu_sc as plsc

N = 2048
E = 32768
D_FEAT = 128
HID = 64
HEADS = 4
DH = 16
WALK = 16
RWSE_DIM = 16
NPAIRS = 4096

_SC_PARAMS = pltpu.CompilerParams(needs_layout_passes=False)


@functools.cache
def _sc_mesh():
    return plsc.VectorSubcoreMesh(core_axis_name="c", subcore_axis_name="s")


ROWS_PER_TILE = 64
COLS_PER_PASS = 1024
ECHUNK = 8192


@functools.cache
def _build_counts_kernel():
    return functools.partial(
        pl.kernel,
        out_type=jax.ShapeDtypeStruct((N, N), jnp.float32),
        mesh=_sc_mesh(),
        scratch_types=[
            pltpu.VMEM((ROWS_PER_TILE, COLS_PER_PASS), jnp.float32),
            pltpu.VMEM((ECHUNK,), jnp.int32),
            pltpu.VMEM((ECHUNK,), jnp.int32),
            pltpu.VMEM((ECHUNK,), jnp.int32),
            pltpu.VMEM((ECHUNK,), jnp.int32),
            pltpu.SemaphoreType.DMA,
            pltpu.SemaphoreType.DMA,
        ],
        compiler_params=_SC_PARAMS,
    )(_build_counts_body)


def _build_counts(row, col):
    return _build_counts_kernel()(row, col)


def _build_counts_body(row_hbm, col_hbm, out_hbm, acc_v, r_a, c_a, r_b, c_b,
                       sem_a, sem_b):
    wid = lax.axis_index("s") * 2 + lax.axis_index("c")
    r0 = wid * ROWS_PER_TILE
    zeros16 = jnp.zeros((16,), jnp.float32)
    ones16 = jnp.ones((16,), jnp.float32)
    nch = E // ECHUNK
    bufs = ((r_a, c_a, sem_a), (r_b, c_b, sem_b))

    def stage(ch):
        rv, cv, sem = bufs[ch % 2]
        h1 = pltpu.async_copy(row_hbm.at[pl.ds(ch * ECHUNK, ECHUNK)], rv, sem)
        h2 = pltpu.async_copy(col_hbm.at[pl.ds(ch * ECHUNK, ECHUNK)], cv, sem)
        return h1, h2

    for p in range(N // COLS_PER_PASS):
        c0 = p * COLS_PER_PASS
        pend = stage(0)

        def zero_body(i, _):
            r = i // 8
            cbase = (i % 8) * 128
            for u in range(8):
                acc_v[r, pl.ds(cbase + u * 16, 16)] = zeros16
            return 0

        lax.fori_loop(0, ROWS_PER_TILE * COLS_PER_PASS // (16 * 8),
                      zero_body, 0)

        for ch in range(nch):
            rv, cv, _ = bufs[ch % 2]
            cur = pend
            if ch + 1 < nch:
                pend = stage(ch + 1)
            cur[0].wait()
            cur[1].wait()

            def group_body(g, _):
                for u in range(4):
                    o = (g * 4 + u) * 16
                    r16 = rv[pl.ds(o, 16)]
                    c16 = cv[pl.ds(o, 16)]
                    m = ((r16 >= r0) & (r16 < r0 + ROWS_PER_TILE)
                         & (c16 >= c0) & (c16 < c0 + COLS_PER_PASS))
                    rr = jnp.where(m, r16 - r0, 0)
                    cc = jnp.where(m, c16 - c0, 0)
                    plsc.addupdate_scatter(acc_v, [rr, cc], ones16, mask=m)
                return 0

            lax.fori_loop(0, ECHUNK // (16 * 4), group_body, 0)

        pltpu.sync_copy(
            acc_v, out_hbm.at[pl.ds(r0, ROWS_PER_TILE), pl.ds(c0, COLS_PER_PASS)])


def _norm_kernel(a_ref, p_ref):
    a = a_ref[...]
    deg = jnp.sum(a, axis=1, keepdims=True)
    p_ref[...] = (a / jnp.maximum(deg, 1.0)).astype(jnp.bfloat16)


def _normalize(a):
    return pl.pallas_call(
        _norm_kernel,
        grid=(8,),
        in_specs=[pl.BlockSpec((N // 8, N), lambda i: (i, 0))],
        out_specs=pl.BlockSpec((N // 8, N), lambda i: (i, 0)),
        out_shape=jax.ShapeDtypeStruct((N, N), jnp.bfloat16),
    )(a)


_MB = 1024


def _mm_kernel(a_ref, b_ref, o_ref, acc_ref):
    @pl.when(pl.program_id(2) == 0)
    def _():
        acc_ref[...] = jnp.zeros_like(acc_ref)

    acc_ref[...] += jnp.dot(a_ref[...], b_ref[...],
                            preferred_element_type=jnp.float32)

    @pl.when(pl.program_id(2) == pl.num_programs(2) - 1)
    def _():
        o_ref[...] = acc_ref[...].astype(jnp.bfloat16)


def _mm(a, b):
    g = N // _MB
    return pl.pallas_call(
        _mm_kernel,
        grid=(g, g, g),
        in_specs=[pl.BlockSpec((_MB, _MB), lambda i, j, k: (i, k)),
                  pl.BlockSpec((_MB, _MB), lambda i, j, k: (k, j))],
        out_specs=pl.BlockSpec((_MB, _MB), lambda i, j, k: (i, j)),
        out_shape=jax.ShapeDtypeStruct((N, N), jnp.bfloat16),
        scratch_shapes=[pltpu.VMEM((_MB, _MB), jnp.float32)],
    )(a, b)


def _mm2_kernel(a_ref, b1_ref, b2_ref, o1_ref, o2_ref, acc1_ref, acc2_ref):
    @pl.when(pl.program_id(2) == 0)
    def _():
        acc1_ref[...] = jnp.zeros_like(acc1_ref)
        acc2_ref[...] = jnp.zeros_like(acc2_ref)

    a = a_ref[...]
    acc1_ref[...] += jnp.dot(a, b1_ref[...],
                             preferred_element_type=jnp.float32)
    acc2_ref[...] += jnp.dot(a, b2_ref[...],
                             preferred_element_type=jnp.float32)

    @pl.when(pl.program_id(2) == pl.num_programs(2) - 1)
    def _():
        o1_ref[...] = acc1_ref[...].astype(jnp.bfloat16)
        o2_ref[...] = acc2_ref[...].astype(jnp.bfloat16)


def _mm2(a, b1, b2):
    g = N // _MB
    sd = jax.ShapeDtypeStruct((N, N), jnp.bfloat16)
    return pl.pallas_call(
        _mm2_kernel,
        grid=(g, g, g),
        in_specs=[pl.BlockSpec((_MB, _MB), lambda i, j, k: (i, k)),
                  pl.BlockSpec((_MB, _MB), lambda i, j, k: (k, j)),
                  pl.BlockSpec((_MB, _MB), lambda i, j, k: (k, j))],
        out_specs=[pl.BlockSpec((_MB, _MB), lambda i, j, k: (i, j)),
                   pl.BlockSpec((_MB, _MB), lambda i, j, k: (i, j))],
        out_shape=[sd, sd],
        scratch_shapes=[pltpu.VMEM((_MB, _MB), jnp.float32),
                        pltpu.VMEM((_MB, _MB), jnp.float32)],
    )(a, b1, b2)


_BT = 128
_DK = 512


def _diag_kernel(p1_ref, p2_ref, p3_ref, p4r_ref, p4c_ref, p8c_ref, p12c_ref,
                 o_ref, acc_ref):
    bi = pl.program_id(0)
    kk = pl.program_id(1)
    nk = pl.num_programs(1)

    @pl.when(kk == 0)
    def _():
        o_ref[...] = jnp.zeros_like(o_ref)
        acc_ref[...] = jnp.zeros_like(acc_ref)

    a_chunks = [p1_ref[...], p2_ref[...], p3_ref[...], p4r_ref[...]]
    b4 = p4c_ref[...]
    b8 = p8c_ref[...]
    b12 = p12c_ref[...]

    def dot(a, b):
        return jnp.dot(a, b, preferred_element_type=jnp.float32)

    for t, a in enumerate(a_chunks[:3]):
        acc_ref[t * _BT:(t + 1) * _BT, :] += dot(a, b4)
        acc_ref[(3 + t) * _BT:(4 + t) * _BT, :] += dot(a, b8)
    for t, a in enumerate(a_chunks):
        acc_ref[(6 + t) * _BT:(7 + t) * _BT, :] += dot(a, b12)

    ii = lax.broadcasted_iota(jnp.int32, (_BT, _BT), 0)
    jj = lax.broadcasted_iota(jnp.int32, (_BT, _BT), 1)
    eye = (ii == jj).astype(jnp.float32)

    @pl.when(kk == bi // 4)
    def _():
        off = (bi % 4) * _BT
        iw = lax.broadcasted_iota(jnp.int32, (_BT, _DK), 0)
        jw = lax.broadcasted_iota(jnp.int32, (_BT, _DK), 1)
        mask_a = (jw == iw + off).astype(jnp.float32)
        it = lax.broadcasted_iota(jnp.int32, (_DK, _BT), 0)
        jt = lax.broadcasted_iota(jnp.int32, (_DK, _BT), 1)
        mask_b = (it == jt + off).astype(jnp.float32)
        z = jnp.zeros((_BT, 1), jnp.float32)

        def rs_a(x):
            return jnp.sum(x.astype(jnp.float32) * mask_a, axis=1,
                           keepdims=True)

        def rs_b(x):
            return jnp.sum(x.astype(jnp.float32) * mask_b, axis=0)[:, None]

        o_ref[...] += jnp.concatenate(
            [rs_a(a_chunks[0]), rs_a(a_chunks[1]), rs_a(a_chunks[2]),
             rs_a(a_chunks[3]), z, z, z, rs_b(b8), z, z, z, rs_b(b12),
             z, z, z, z], axis=1)

    @pl.when(kk == nk - 1)
    def _():
        acc = acc_ref[...]

        def dg(t):
            sub = acc[t * _BT:(t + 1) * _BT, :]
            return jnp.sum(sub * eye, axis=1, keepdims=True)

        z = jnp.zeros((_BT, 1), jnp.float32)
        o_ref[...] += jnp.concatenate(
            [z, z, z, z, dg(0), dg(1), dg(2), z, dg(3), dg(4), dg(5), z,
             dg(6), dg(7), dg(8), dg(9)], axis=1)


def _diag_pairs(p1, p2, p3, p4, p8, p12):
    row = pl.BlockSpec((_BT, _DK), lambda i, k: (i, k))
    col = pl.BlockSpec((_DK, _BT), lambda i, k: (k, i))
    return pl.pallas_call(
        _diag_kernel,
        grid=(N // _BT, N // _DK),
        in_specs=[row, row, row, row, col, col, col],
        out_specs=pl.BlockSpec((_BT, WALK), lambda i, k: (i, 0)),
        out_shape=jax.ShapeDtypeStruct((N, WALK), jnp.float32),
        scratch_shapes=[pltpu.VMEM((10 * _BT, _BT), jnp.float32)],
    )(p1, p2, p3, p4, p4, p8, p12)


_RB = 256


def _cdot(a, b):
    return lax.dot_general(a, b, (((1,), (1,)), ((), ())),
                           preferred_element_type=jnp.float32)


def _inproj_kernel(x_ref, dg_ref, wr_ref, br_ref, w1_ref, w2_ref, b_ref,
                   wc_ref, bc_ref, h_ref, q_ref):
    pe = _cdot(dg_ref[...], wr_ref[...]) + br_ref[...]
    h = _cdot(x_ref[...], w1_ref[...]) + _cdot(pe, w2_ref[...]) + b_ref[...]
    h_ref[...] = h
    q_ref[...] = _cdot(h, wc_ref[...]) + bc_ref[...]


def _input_proj(x, diags, w_rwse, b_rwse, w1, w2, b_in, wcat, bcat):
    return pl.pallas_call(
        _inproj_kernel,
        grid=(N // _RB,),
        in_specs=[
            pl.BlockSpec((_RB, D_FEAT), lambda i: (i, 0)),
            pl.BlockSpec((_RB, WALK), lambda i: (i, 0)),
            pl.BlockSpec((RWSE_DIM, WALK), lambda i: (0, 0)),
            pl.BlockSpec((1, RWSE_DIM), lambda i: (0, 0)),
            pl.BlockSpec((HID, D_FEAT), lambda i: (0, 0)),
            pl.BlockSpec((HID, RWSE_DIM), lambda i: (0, 0)),
            pl.BlockSpec((1, HID), lambda i: (0, 0)),
            pl.BlockSpec((4 * HID, HID), lambda i: (0, 0)),
            pl.BlockSpec((1, 4 * HID), lambda i: (0, 0)),
        ],
        out_specs=[pl.BlockSpec((_RB, HID), lambda i: (i, 0)),
                   pl.BlockSpec((_RB, 4 * HID), lambda i: (i, 0))],
        out_shape=[jax.ShapeDtypeStruct((N, HID), jnp.float32),
                   jax.ShapeDtypeStruct((N, 4 * HID), jnp.float32)],
    )(x, diags, w_rwse, b_rwse, w1, w2, b_in, wcat, bcat)


_BA = 512


def _attn_accum(q_ref, kv_ref, a_ref, num_ref, den_ref):
    @pl.when(pl.program_id(1) == 0)
    def _():
        num_ref[...] = jnp.zeros_like(num_ref)
        den_ref[...] = jnp.zeros_like(den_ref)

    at = a_ref[...]
    for h in range(HEADS):
        sl = slice(h * DH, (h + 1) * DH)
        ksl = slice(HID + h * DH, HID + (h + 1) * DH)
        vsl = slice(2 * HID + h * DH, 2 * HID + (h + 1) * DH)
        st = _cdot(kv_ref[:, ksl], q_ref[:, sl])
        w = at * jnp.exp(st * 0.25)
        num_ref[:, sl] += lax.dot_general(
            w, kv_ref[:, vsl], (((0,), (0,)), ((), ())),
            preferred_element_type=jnp.float32)
        den_ref[:, h:h + 1] += jnp.sum(w, axis=0)[:, None]


def _conv_core(h_ref, s_ref, num, den, g_ref, b_ref, rows):
    denr = jnp.concatenate(
        [jnp.broadcast_to(den[:, h:h + 1], (rows, DH)) for h in range(HEADS)],
        axis=1)
    conv = num / (denr + 1e-16) + s_ref[:, 3 * HID:4 * HID]
    z = h_ref[...] + conv
    mu = jnp.mean(z, axis=1, keepdims=True)
    zc = z - mu
    var = jnp.mean(zc * zc, axis=1, keepdims=True)
    hn = zc / jnp.sqrt(var + 1e-5) * g_ref[...] + b_ref[...]
    return jnp.maximum(hn, 0.0)


def _attn_proj_kernel(q_ref, kv_ref, a_ref, h_ref, g_ref, b_ref, wc_ref,
                      bc_ref, o_ref, qo_ref, num_ref, den_ref):
    _attn_accum(q_ref, kv_ref, a_ref, num_ref, den_ref)

    @pl.when(pl.program_id(1) == pl.num_programs(1) - 1)
    def _():
        res = _conv_core(h_ref, q_ref, num_ref[...], den_ref[...], g_ref,
                         b_ref, _BA)
        o_ref[...] = res
        qo_ref[...] = _cdot(res, wc_ref[...]) + bc_ref[...]


def _attn_pad_kernel(q_ref, kv_ref, a_ref, h_ref, g_ref, b_ref, o_ref,
                     num_ref, den_ref):
    _attn_accum(q_ref, kv_ref, a_ref, num_ref, den_ref)

    @pl.when(pl.program_id(1) == pl.num_programs(1) - 1)
    def _():
        res = _conv_core(h_ref, q_ref, num_ref[...], den_ref[...], g_ref,
                         b_ref, _BA)
        o_ref[...] = jnp.concatenate([res, jnp.zeros_like(res)], axis=1)


_ATTN_IN = [
    pl.BlockSpec((_BA, 4 * HID), lambda i, j: (i, 0)),
    pl.BlockSpec((_BA, 4 * HID), lambda i, j: (j, 0)),
    pl.BlockSpec((_BA, _BA), lambda i, j: (j, i)),
    pl.BlockSpec((_BA, HID), lambda i, j: (i, 0)),
    pl.BlockSpec((1, HID), lambda i, j: (0, 0)),
    pl.BlockSpec((1, HID), lambda i, j: (0, 0)),
]
_ATTN_SCRATCH = [pltpu.VMEM((_BA, HID), jnp.float32),
                 pltpu.VMEM((_BA, HEADS), jnp.float32)]


def _attn_proj(qkvs, a, h, g, be, wcat, bcat):
    gr = N // _BA
    return pl.pallas_call(
        _attn_proj_kernel,
        grid=(gr, gr),
        in_specs=_ATTN_IN + [
            pl.BlockSpec((4 * HID, HID), lambda i, j: (0, 0)),
            pl.BlockSpec((1, 4 * HID), lambda i, j: (0, 0)),
        ],
        out_specs=[pl.BlockSpec((_BA, HID), lambda i, j: (i, 0)),
                   pl.BlockSpec((_BA, 4 * HID), lambda i, j: (i, 0))],
        out_shape=[jax.ShapeDtypeStruct((N, HID), jnp.float32),
                   jax.ShapeDtypeStruct((N, 4 * HID), jnp.float32)],
        scratch_shapes=_ATTN_SCRATCH,
        compiler_params=pltpu.CompilerParams(
            fuse_transposed_lhs_in_matmul=True),
    )(qkvs, qkvs, a, h, g, be, wcat, bcat)


def _attn_pad(qkvs, a, h, g, be):
    gr = N // _BA
    return pl.pallas_call(
        _attn_pad_kernel,
        grid=(gr, gr),
        in_specs=_ATTN_IN,
        out_specs=pl.BlockSpec((_BA, 2 * HID), lambda i, j: (i, 0)),
        out_shape=jax.ShapeDtypeStruct((N, 2 * HID), jnp.float32),
        scratch_shapes=_ATTN_SCRATCH,
        compiler_params=pltpu.CompilerParams(
            fuse_transposed_lhs_in_matmul=True),
    )(qkvs, qkvs, a, h, g, be)


PAIRS_PER = NPAIRS // 32


@functools.cache
def _pairs_kernel():
    return functools.partial(
        pl.kernel,
        out_type=jax.ShapeDtypeStruct((NPAIRS,), jnp.float32),
        mesh=_sc_mesh(),
        scratch_types=[
            pltpu.VMEM((PAIRS_PER,), jnp.int32),
            pltpu.VMEM((PAIRS_PER,), jnp.int32),
            pltpu.VMEM((PAIRS_PER, 2 * HID), jnp.float32),
            pltpu.VMEM((PAIRS_PER, 2 * HID), jnp.float32),
            pltpu.VMEM((PAIRS_PER,), jnp.float32),
            pltpu.SemaphoreType.DMA,
        ],
        compiler_params=_SC_PARAMS,
    )(_pairs_body)


def _pairs(h, src, dst):
    return _pairs_kernel()(h, src, dst)


def _pairs_body(h_hbm, src_hbm, dst_hbm, out_hbm, s_v, d_v, hs_v, hd_v, res_v,
                sem):
    wid = lax.axis_index("s") * 2 + lax.axis_index("c")
    base = wid * PAIRS_PER
    pltpu.sync_copy(src_hbm.at[pl.ds(base, PAIRS_PER)], s_v)
    pltpu.sync_copy(dst_hbm.at[pl.ds(base, PAIRS_PER)], d_v)
    pltpu.async_copy(h_hbm.at[s_v], hs_v, sem).wait()
    pltpu.async_copy(h_hbm.at[d_v], hd_v, sem).wait()

    def group_body(g, _):
        rows = g * 16 + lax.iota(jnp.int32, 16)
        acc = jnp.zeros((16,), jnp.float32)
        for c in range(HID):
            colv = jnp.full((16,), c, jnp.int32)
            acc = acc + (plsc.load_gather(hs_v, [rows, colv])
                         * plsc.load_gather(hd_v, [rows, colv]))
        res_v[pl.ds(g * 16, 16)] = 1.0 / (1.0 + jnp.exp(-acc))
        return 0

    lax.fori_loop(0, PAIRS_PER // 16, group_body, 0)
    pltpu.sync_copy(res_v, out_hbm.at[pl.ds(base, PAIRS_PER)])


def kernel(x, edge_index, src, dst, W_rwse, b_rwse, W_in, b_in,
           Wq0, bq0, Wk0, bk0, Wv0, bv0, Ws0, bs0, g0, be0,
           Wq1, bq1, Wk1, bk1, Wv1, bv1, Ws1, bs1, g1, be1):
    row = edge_index[0]
    col = edge_index[1]
    a = _build_counts(row, col)
    p = _normalize(a)
    p2 = _mm(p, p)
    p3, p4 = _mm2(p2, p, p2)
    p8 = _mm(p4, p4)
    p12 = _mm(p8, p4)
    diags = _diag_pairs(p, p2, p3, p4, p8, p12)
    wcat0 = jnp.concatenate([Wq0, Wk0, Wv0, Ws0], axis=0)
    bcat0 = jnp.concatenate([bq0, bk0, bv0, bs0]).reshape(1, -1)
    wcat1 = jnp.concatenate([Wq1, Wk1, Wv1, Ws1], axis=0)
    bcat1 = jnp.concatenate([bq1, bk1, bv1, bs1]).reshape(1, -1)
    h, qkvs0 = _input_proj(x, diags, W_rwse, b_rwse.reshape(1, -1),
                           W_in[:, :D_FEAT], W_in[:, D_FEAT:],
                           b_in.reshape(1, -1), wcat0, bcat0)
    h, qkvs1 = _attn_proj(qkvs0, a, h, g0.reshape(1, -1),
                          be0.reshape(1, -1), wcat1, bcat1)
    hp = _attn_pad(qkvs1, a, h, g1.reshape(1, -1), be1.reshape(1, -1))
    return _pairs(hp, src, dst)


# speedup vs baseline: 20.8316x; 1.0237x over previous
import functools

import jax
import jax.numpy as jnp
from jax import lax
from jax.experimental import pallas as pl
from jax.experimental.pallas import tpu as pltpu
from jax.experimental.pallas import tpu_sc as plsc

N = 2048
E = 32768
D_FEAT = 128
HID = 64
HEADS = 4
DH = 16
WALK = 16
RWSE_DIM = 16
NPAIRS = 4096

_SC_PARAMS = pltpu.CompilerParams(needs_layout_passes=False)


@functools.cache
def _sc_mesh():
    return plsc.VectorSubcoreMesh(core_axis_name="c", subcore_axis_name="s")


ROWS_PER_TILE = 64
COLS_PER_PASS = 1024
ECHUNK = 8192


@functools.cache
def _build_counts_kernel():
    return functools.partial(
        pl.kernel,
        out_type=jax.ShapeDtypeStruct((N, N), jnp.float32),
        mesh=_sc_mesh(),
        scratch_types=[
            pltpu.VMEM((ROWS_PER_TILE, COLS_PER_PASS), jnp.float32),
            pltpu.VMEM((ECHUNK,), jnp.int32),
            pltpu.VMEM((ECHUNK,), jnp.int32),
            pltpu.VMEM((ECHUNK,), jnp.int32),
            pltpu.VMEM((ECHUNK,), jnp.int32),
            pltpu.SemaphoreType.DMA,
            pltpu.SemaphoreType.DMA,
        ],
        compiler_params=_SC_PARAMS,
    )(_build_counts_body)


def _build_counts(row, col):
    return _build_counts_kernel()(row, col)


def _build_counts_body(row_hbm, col_hbm, out_hbm, acc_v, r_a, c_a, r_b, c_b,
                       sem_a, sem_b):
    wid = lax.axis_index("s") * 2 + lax.axis_index("c")
    r0 = wid * ROWS_PER_TILE
    zeros16 = jnp.zeros((16,), jnp.float32)
    ones16 = jnp.ones((16,), jnp.float32)
    nch = E // ECHUNK
    bufs = ((r_a, c_a, sem_a), (r_b, c_b, sem_b))

    def stage(ch):
        rv, cv, sem = bufs[ch % 2]
        h1 = pltpu.async_copy(row_hbm.at[pl.ds(ch * ECHUNK, ECHUNK)], rv, sem)
        h2 = pltpu.async_copy(col_hbm.at[pl.ds(ch * ECHUNK, ECHUNK)], cv, sem)
        return h1, h2

    for p in range(N // COLS_PER_PASS):
        c0 = p * COLS_PER_PASS
        pend = stage(0)

        def zero_body(i, _):
            r = i // 8
            cbase = (i % 8) * 128
            for u in range(8):
                acc_v[r, pl.ds(cbase + u * 16, 16)] = zeros16
            return 0

        lax.fori_loop(0, ROWS_PER_TILE * COLS_PER_PASS // (16 * 8),
                      zero_body, 0)

        for ch in range(nch):
            rv, cv, _ = bufs[ch % 2]
            cur = pend
            if ch + 1 < nch:
                pend = stage(ch + 1)
            cur[0].wait()
            cur[1].wait()

            def group_body(g, _):
                for u in range(4):
                    o = (g * 4 + u) * 16
                    r16 = rv[pl.ds(o, 16)]
                    c16 = cv[pl.ds(o, 16)]
                    m = ((r16 >= r0) & (r16 < r0 + ROWS_PER_TILE)
                         & (c16 >= c0) & (c16 < c0 + COLS_PER_PASS))
                    rr = jnp.where(m, r16 - r0, 0)
                    cc = jnp.where(m, c16 - c0, 0)
                    plsc.addupdate_scatter(acc_v, [rr, cc], ones16, mask=m)
                return 0

            lax.fori_loop(0, ECHUNK // (16 * 4), group_body, 0)

        pltpu.sync_copy(
            acc_v, out_hbm.at[pl.ds(r0, ROWS_PER_TILE), pl.ds(c0, COLS_PER_PASS)])


def _norm_kernel(a_ref, p_ref):
    a = a_ref[...]
    deg = jnp.sum(a, axis=1, keepdims=True)
    p_ref[...] = (a / jnp.maximum(deg, 1.0)).astype(jnp.bfloat16)


def _normalize(a):
    return pl.pallas_call(
        _norm_kernel,
        grid=(8,),
        in_specs=[pl.BlockSpec((N // 8, N), lambda i: (i, 0))],
        out_specs=pl.BlockSpec((N // 8, N), lambda i: (i, 0)),
        out_shape=jax.ShapeDtypeStruct((N, N), jnp.bfloat16),
    )(a)


_MB = 1024


def _mm_kernel(a_ref, b_ref, o_ref, acc_ref):
    @pl.when(pl.program_id(2) == 0)
    def _():
        acc_ref[...] = jnp.zeros_like(acc_ref)

    acc_ref[...] += jnp.dot(a_ref[...], b_ref[...],
                            preferred_element_type=jnp.float32)

    @pl.when(pl.program_id(2) == pl.num_programs(2) - 1)
    def _():
        o_ref[...] = acc_ref[...].astype(jnp.bfloat16)


def _mm(a, b):
    g = N // _MB
    return pl.pallas_call(
        _mm_kernel,
        grid=(g, g, g),
        in_specs=[pl.BlockSpec((_MB, _MB), lambda i, j, k: (i, k)),
                  pl.BlockSpec((_MB, _MB), lambda i, j, k: (k, j))],
        out_specs=pl.BlockSpec((_MB, _MB), lambda i, j, k: (i, j)),
        out_shape=jax.ShapeDtypeStruct((N, N), jnp.bfloat16),
        scratch_shapes=[pltpu.VMEM((_MB, _MB), jnp.float32)],
    )(a, b)


def _mm2_kernel(a_ref, b1_ref, b2_ref, o1_ref, o2_ref, acc1_ref, acc2_ref):
    @pl.when(pl.program_id(2) == 0)
    def _():
        acc1_ref[...] = jnp.zeros_like(acc1_ref)
        acc2_ref[...] = jnp.zeros_like(acc2_ref)

    a = a_ref[...]
    acc1_ref[...] += jnp.dot(a, b1_ref[...],
                             preferred_element_type=jnp.float32)
    acc2_ref[...] += jnp.dot(a, b2_ref[...],
                             preferred_element_type=jnp.float32)

    @pl.when(pl.program_id(2) == pl.num_programs(2) - 1)
    def _():
        o1_ref[...] = acc1_ref[...].astype(jnp.bfloat16)
        o2_ref[...] = acc2_ref[...].astype(jnp.bfloat16)


def _mm2(a, b1, b2):
    g = N // _MB
    sd = jax.ShapeDtypeStruct((N, N), jnp.bfloat16)
    return pl.pallas_call(
        _mm2_kernel,
        grid=(g, g, g),
        in_specs=[pl.BlockSpec((_MB, _MB), lambda i, j, k: (i, k)),
                  pl.BlockSpec((_MB, _MB), lambda i, j, k: (k, j)),
                  pl.BlockSpec((_MB, _MB), lambda i, j, k: (k, j))],
        out_specs=[pl.BlockSpec((_MB, _MB), lambda i, j, k: (i, j)),
                   pl.BlockSpec((_MB, _MB), lambda i, j, k: (i, j))],
        out_shape=[sd, sd],
        scratch_shapes=[pltpu.VMEM((_MB, _MB), jnp.float32),
                        pltpu.VMEM((_MB, _MB), jnp.float32)],
    )(a, b1, b2)


_BT = 128
_DK = 512


def _diag_kernel(p1_ref, p2_ref, p3_ref, p4r_ref, p4c_ref, p8c_ref, p12c_ref,
                 o_ref, acc_ref):
    bi = pl.program_id(0)
    kk = pl.program_id(1)
    nk = pl.num_programs(1)

    @pl.when(kk == 0)
    def _():
        o_ref[...] = jnp.zeros_like(o_ref)
        acc_ref[...] = jnp.zeros_like(acc_ref)

    a_chunks = [p1_ref[...], p2_ref[...], p3_ref[...], p4r_ref[...]]
    b4 = p4c_ref[...]
    b8 = p8c_ref[...]
    b12 = p12c_ref[...]

    def dot(a, b):
        return jnp.dot(a, b, preferred_element_type=jnp.float32)

    for t, a in enumerate(a_chunks[:3]):
        acc_ref[t * _BT:(t + 1) * _BT, :] += dot(a, b4)
        acc_ref[(3 + t) * _BT:(4 + t) * _BT, :] += dot(a, b8)
    for t, a in enumerate(a_chunks):
        acc_ref[(6 + t) * _BT:(7 + t) * _BT, :] += dot(a, b12)

    ii = lax.broadcasted_iota(jnp.int32, (_BT, _BT), 0)
    jj = lax.broadcasted_iota(jnp.int32, (_BT, _BT), 1)
    eye = (ii == jj).astype(jnp.float32)

    @pl.when(kk == bi // 4)
    def _():
        off = (bi % 4) * _BT
        iw = lax.broadcasted_iota(jnp.int32, (_BT, _DK), 0)
        jw = lax.broadcasted_iota(jnp.int32, (_BT, _DK), 1)
        mask_a = (jw == iw + off).astype(jnp.float32)
        it = lax.broadcasted_iota(jnp.int32, (_DK, _BT), 0)
        jt = lax.broadcasted_iota(jnp.int32, (_DK, _BT), 1)
        mask_b = (it == jt + off).astype(jnp.float32)
        z = jnp.zeros((_BT, 1), jnp.float32)

        def rs_a(x):
            return jnp.sum(x.astype(jnp.float32) * mask_a, axis=1,
                           keepdims=True)

        def rs_b(x):
            return jnp.sum(x.astype(jnp.float32) * mask_b, axis=0)[:, None]

        o_ref[...] += jnp.concatenate(
            [rs_a(a_chunks[0]), rs_a(a_chunks[1]), rs_a(a_chunks[2]),
             rs_a(a_chunks[3]), z, z, z, rs_b(b8), z, z, z, rs_b(b12),
             z, z, z, z], axis=1)

    @pl.when(kk == nk - 1)
    def _():
        acc = acc_ref[...]

        def dg(t):
            sub = acc[t * _BT:(t + 1) * _BT, :]
            return jnp.sum(sub * eye, axis=1, keepdims=True)

        z = jnp.zeros((_BT, 1), jnp.float32)
        o_ref[...] += jnp.concatenate(
            [z, z, z, z, dg(0), dg(1), dg(2), z, dg(3), dg(4), dg(5), z,
             dg(6), dg(7), dg(8), dg(9)], axis=1)


def _diag_pairs(p1, p2, p3, p4, p8, p12):
    row = pl.BlockSpec((_BT, _DK), lambda i, k: (i, k))
    col = pl.BlockSpec((_DK, _BT), lambda i, k: (k, i))
    return pl.pallas_call(
        _diag_kernel,
        grid=(N // _BT, N // _DK),
        in_specs=[row, row, row, row, col, col, col],
        out_specs=pl.BlockSpec((_BT, WALK), lambda i, k: (i, 0)),
        out_shape=jax.ShapeDtypeStruct((N, WALK), jnp.float32),
        scratch_shapes=[pltpu.VMEM((10 * _BT, _BT), jnp.float32)],
    )(p1, p2, p3, p4, p4, p8, p12)


_RB = 256


def _cdot(a, b):
    return lax.dot_general(a, b, (((1,), (1,)), ((), ())),
                           preferred_element_type=jnp.float32)


def _inproj_kernel(x_ref, dg_ref, wr_ref, br_ref, w1_ref, w2_ref, b_ref,
                   wc_ref, bc_ref, h_ref, q_ref):
    pe = _cdot(dg_ref[...], wr_ref[...]) + br_ref[...]
    h = _cdot(x_ref[...], w1_ref[...]) + _cdot(pe, w2_ref[...]) + b_ref[...]
    h_ref[...] = h
    q_ref[...] = _cdot(h, wc_ref[...]) + bc_ref[...]


def _input_proj(x, diags, w_rwse, b_rwse, w1, w2, b_in, wcat, bcat):
    return pl.pallas_call(
        _inproj_kernel,
        grid=(N // _RB,),
        in_specs=[
            pl.BlockSpec((_RB, D_FEAT), lambda i: (i, 0)),
            pl.BlockSpec((_RB, WALK), lambda i: (i, 0)),
            pl.BlockSpec((RWSE_DIM, WALK), lambda i: (0, 0)),
            pl.BlockSpec((1, RWSE_DIM), lambda i: (0, 0)),
            pl.BlockSpec((HID, D_FEAT), lambda i: (0, 0)),
            pl.BlockSpec((HID, RWSE_DIM), lambda i: (0, 0)),
            pl.BlockSpec((1, HID), lambda i: (0, 0)),
            pl.BlockSpec((4 * HID, HID), lambda i: (0, 0)),
            pl.BlockSpec((1, 4 * HID), lambda i: (0, 0)),
        ],
        out_specs=[pl.BlockSpec((_RB, HID), lambda i: (i, 0)),
                   pl.BlockSpec((_RB, 4 * HID), lambda i: (i, 0))],
        out_shape=[jax.ShapeDtypeStruct((N, HID), jnp.float32),
                   jax.ShapeDtypeStruct((N, 4 * HID), jnp.float32)],
    )(x, diags, w_rwse, b_rwse, w1, w2, b_in, wcat, bcat)


_BA = 1024


def _attn_accum(q_ref, kv_ref, a_ref, num_ref, den_ref):
    @pl.when(pl.program_id(1) == 0)
    def _():
        num_ref[...] = jnp.zeros_like(num_ref)
        den_ref[...] = jnp.zeros_like(den_ref)

    at = a_ref[...]
    for h in range(HEADS):
        sl = slice(h * DH, (h + 1) * DH)
        ksl = slice(HID + h * DH, HID + (h + 1) * DH)
        vsl = slice(2 * HID + h * DH, 2 * HID + (h + 1) * DH)
        st = _cdot(kv_ref[:, ksl], q_ref[:, sl])
        w = at * jnp.exp(st * 0.25)
        num_ref[:, sl] += lax.dot_general(
            w, kv_ref[:, vsl], (((0,), (0,)), ((), ())),
            preferred_element_type=jnp.float32)
        den_ref[:, h:h + 1] += jnp.sum(w, axis=0)[:, None]


def _conv_core(h_ref, s_ref, num, den, g_ref, b_ref, rows):
    denr = jnp.concatenate(
        [jnp.broadcast_to(den[:, h:h + 1], (rows, DH)) for h in range(HEADS)],
        axis=1)
    conv = num / (denr + 1e-16) + s_ref[:, 3 * HID:4 * HID]
    z = h_ref[...] + conv
    mu = jnp.mean(z, axis=1, keepdims=True)
    zc = z - mu
    var = jnp.mean(zc * zc, axis=1, keepdims=True)
    hn = zc / jnp.sqrt(var + 1e-5) * g_ref[...] + b_ref[...]
    return jnp.maximum(hn, 0.0)


def _attn_proj_kernel(q_ref, kv_ref, a_ref, h_ref, g_ref, b_ref, wc_ref,
                      bc_ref, o_ref, qo_ref, num_ref, den_ref):
    _attn_accum(q_ref, kv_ref, a_ref, num_ref, den_ref)

    @pl.when(pl.program_id(1) == pl.num_programs(1) - 1)
    def _():
        res = _conv_core(h_ref, q_ref, num_ref[...], den_ref[...], g_ref,
                         b_ref, _BA)
        o_ref[...] = res
        qo_ref[...] = _cdot(res, wc_ref[...]) + bc_ref[...]


def _attn_pad_kernel(q_ref, kv_ref, a_ref, h_ref, g_ref, b_ref, o_ref,
                     num_ref, den_ref):
    _attn_accum(q_ref, kv_ref, a_ref, num_ref, den_ref)

    @pl.when(pl.program_id(1) == pl.num_programs(1) - 1)
    def _():
        res = _conv_core(h_ref, q_ref, num_ref[...], den_ref[...], g_ref,
                         b_ref, _BA)
        o_ref[...] = jnp.concatenate([res, jnp.zeros_like(res)], axis=1)


_ATTN_IN = [
    pl.BlockSpec((_BA, 4 * HID), lambda i, j: (i, 0)),
    pl.BlockSpec((_BA, 4 * HID), lambda i, j: (j, 0)),
    pl.BlockSpec((_BA, _BA), lambda i, j: (j, i)),
    pl.BlockSpec((_BA, HID), lambda i, j: (i, 0)),
    pl.BlockSpec((1, HID), lambda i, j: (0, 0)),
    pl.BlockSpec((1, HID), lambda i, j: (0, 0)),
]
_ATTN_SCRATCH = [pltpu.VMEM((_BA, HID), jnp.float32),
                 pltpu.VMEM((_BA, HEADS), jnp.float32)]


def _attn_proj(qkvs, a, h, g, be, wcat, bcat):
    gr = N // _BA
    return pl.pallas_call(
        _attn_proj_kernel,
        grid=(gr, gr),
        in_specs=_ATTN_IN + [
            pl.BlockSpec((4 * HID, HID), lambda i, j: (0, 0)),
            pl.BlockSpec((1, 4 * HID), lambda i, j: (0, 0)),
        ],
        out_specs=[pl.BlockSpec((_BA, HID), lambda i, j: (i, 0)),
                   pl.BlockSpec((_BA, 4 * HID), lambda i, j: (i, 0))],
        out_shape=[jax.ShapeDtypeStruct((N, HID), jnp.float32),
                   jax.ShapeDtypeStruct((N, 4 * HID), jnp.float32)],
        scratch_shapes=_ATTN_SCRATCH,
        compiler_params=pltpu.CompilerParams(
            fuse_transposed_lhs_in_matmul=True),
    )(qkvs, qkvs, a, h, g, be, wcat, bcat)


def _attn_pad(qkvs, a, h, g, be):
    gr = N // _BA
    return pl.pallas_call(
        _attn_pad_kernel,
        grid=(gr, gr),
        in_specs=_ATTN_IN,
        out_specs=pl.BlockSpec((_BA, 2 * HID), lambda i, j: (i, 0)),
        out_shape=jax.ShapeDtypeStruct((N, 2 * HID), jnp.float32),
        scratch_shapes=_ATTN_SCRATCH,
        compiler_params=pltpu.CompilerParams(
            fuse_transposed_lhs_in_matmul=True),
    )(qkvs, qkvs, a, h, g, be)


PAIRS_PER = NPAIRS // 32


@functools.cache
def _pairs_kernel():
    return functools.partial(
        pl.kernel,
        out_type=jax.ShapeDtypeStruct((NPAIRS,), jnp.float32),
        mesh=_sc_mesh(),
        scratch_types=[
            pltpu.VMEM((PAIRS_PER,), jnp.int32),
            pltpu.VMEM((PAIRS_PER,), jnp.int32),
            pltpu.VMEM((PAIRS_PER, 2 * HID), jnp.float32),
            pltpu.VMEM((PAIRS_PER, 2 * HID), jnp.float32),
            pltpu.VMEM((PAIRS_PER,), jnp.float32),
            pltpu.SemaphoreType.DMA,
        ],
        compiler_params=_SC_PARAMS,
    )(_pairs_body)


def _pairs(h, src, dst):
    return _pairs_kernel()(h, src, dst)


def _pairs_body(h_hbm, src_hbm, dst_hbm, out_hbm, s_v, d_v, hs_v, hd_v, res_v,
                sem):
    wid = lax.axis_index("s") * 2 + lax.axis_index("c")
    base = wid * PAIRS_PER
    pltpu.sync_copy(src_hbm.at[pl.ds(base, PAIRS_PER)], s_v)
    pltpu.sync_copy(dst_hbm.at[pl.ds(base, PAIRS_PER)], d_v)
    pltpu.async_copy(h_hbm.at[s_v], hs_v, sem).wait()
    pltpu.async_copy(h_hbm.at[d_v], hd_v, sem).wait()

    def group_body(g, _):
        rows = g * 16 + lax.iota(jnp.int32, 16)
        acc = jnp.zeros((16,), jnp.float32)
        for c in range(HID):
            colv = jnp.full((16,), c, jnp.int32)
            acc = acc + (plsc.load_gather(hs_v, [rows, colv])
                         * plsc.load_gather(hd_v, [rows, colv]))
        res_v[pl.ds(g * 16, 16)] = 1.0 / (1.0 + jnp.exp(-acc))
        return 0

    lax.fori_loop(0, PAIRS_PER // 16, group_body, 0)
    pltpu.sync_copy(res_v, out_hbm.at[pl.ds(base, PAIRS_PER)])


def kernel(x, edge_index, src, dst, W_rwse, b_rwse, W_in, b_in,
           Wq0, bq0, Wk0, bk0, Wv0, bv0, Ws0, bs0, g0, be0,
           Wq1, bq1, Wk1, bk1, Wv1, bv1, Ws1, bs1, g1, be1):
    row = edge_index[0]
    col = edge_index[1]
    a = _build_counts(row, col)
    p = _normalize(a)
    p2 = _mm(p, p)
    p3, p4 = _mm2(p2, p, p2)
    p8 = _mm(p4, p4)
    p12 = _mm(p8, p4)
    diags = _diag_pairs(p, p2, p3, p4, p8, p12)
    wcat0 = jnp.concatenate([Wq0, Wk0, Wv0, Ws0], axis=0)
    bcat0 = jnp.concatenate([bq0, bk0, bv0, bs0]).reshape(1, -1)
    wcat1 = jnp.concatenate([Wq1, Wk1, Wv1, Ws1], axis=0)
    bcat1 = jnp.concatenate([bq1, bk1, bv1, bs1]).reshape(1, -1)
    h, qkvs0 = _input_proj(x, diags, W_rwse, b_rwse.reshape(1, -1),
                           W_in[:, :D_FEAT], W_in[:, D_FEAT:],
                           b_in.reshape(1, -1), wcat0, bcat0)
    h, qkvs1 = _attn_proj(qkvs0, a, h, g0.reshape(1, -1),
                          be0.reshape(1, -1), wcat1, bcat1)
    hp = _attn_pad(qkvs1, a, h, g1.reshape(1, -1), be1.reshape(1, -1))
    return _pairs(hp, src, dst)


# speedup vs baseline: 24.5735x; 1.1796x over previous
import functools

import jax
import jax.numpy as jnp
from jax import lax
from jax.experimental import pallas as pl
from jax.experimental.pallas import tpu as pltpu
from jax.experimental.pallas import tpu_sc as plsc

N = 2048
E = 32768
D_FEAT = 128
HID = 64
HEADS = 4
DH = 16
WALK = 16
RWSE_DIM = 16
NPAIRS = 4096

_SC_PARAMS = pltpu.CompilerParams(needs_layout_passes=False)


@functools.cache
def _sc_mesh():
    return plsc.VectorSubcoreMesh(core_axis_name="c", subcore_axis_name="s")


ROWS_PER_TILE = 64
COLS_PER_PASS = 1024
ECHUNK = 8192


@functools.cache
def _build_counts_kernel():
    return functools.partial(
        pl.kernel,
        out_type=jax.ShapeDtypeStruct((N, N), jnp.float32),
        mesh=_sc_mesh(),
        scratch_types=[
            pltpu.VMEM((ROWS_PER_TILE, COLS_PER_PASS), jnp.float32),
            pltpu.VMEM((ECHUNK,), jnp.int32),
            pltpu.VMEM((ECHUNK,), jnp.int32),
            pltpu.VMEM((ECHUNK,), jnp.int32),
            pltpu.VMEM((ECHUNK,), jnp.int32),
            pltpu.SemaphoreType.DMA,
            pltpu.SemaphoreType.DMA,
        ],
        compiler_params=_SC_PARAMS,
    )(_build_counts_body)


def _build_counts(row, col):
    return _build_counts_kernel()(row, col)


def _build_counts_body(row_hbm, col_hbm, out_hbm, acc_v, r_a, c_a, r_b, c_b,
                       sem_a, sem_b):
    wid = lax.axis_index("s") * 2 + lax.axis_index("c")
    r0 = wid * ROWS_PER_TILE
    zeros16 = jnp.zeros((16,), jnp.float32)
    ones16 = jnp.ones((16,), jnp.float32)
    nch = E // ECHUNK
    bufs = ((r_a, c_a, sem_a), (r_b, c_b, sem_b))

    def stage(ch):
        rv, cv, sem = bufs[ch % 2]
        h1 = pltpu.async_copy(row_hbm.at[pl.ds(ch * ECHUNK, ECHUNK)], rv, sem)
        h2 = pltpu.async_copy(col_hbm.at[pl.ds(ch * ECHUNK, ECHUNK)], cv, sem)
        return h1, h2

    for p in range(N // COLS_PER_PASS):
        c0 = p * COLS_PER_PASS
        pend = stage(0)

        def zero_body(i, _):
            r = i // 8
            cbase = (i % 8) * 128
            for u in range(8):
                acc_v[r, pl.ds(cbase + u * 16, 16)] = zeros16
            return 0

        lax.fori_loop(0, ROWS_PER_TILE * COLS_PER_PASS // (16 * 8),
                      zero_body, 0)

        for ch in range(nch):
            rv, cv, _ = bufs[ch % 2]
            cur = pend
            if ch + 1 < nch:
                pend = stage(ch + 1)
            cur[0].wait()
            cur[1].wait()

            def group_body(g, _):
                for u in range(4):
                    o = (g * 4 + u) * 16
                    r16 = rv[pl.ds(o, 16)]
                    c16 = cv[pl.ds(o, 16)]
                    m = ((r16 >= r0) & (r16 < r0 + ROWS_PER_TILE)
                         & (c16 >= c0) & (c16 < c0 + COLS_PER_PASS))
                    rr = jnp.where(m, r16 - r0, 0)
                    cc = jnp.where(m, c16 - c0, 0)
                    plsc.addupdate_scatter(acc_v, [rr, cc], ones16, mask=m)
                return 0

            lax.fori_loop(0, ECHUNK // (16 * 4), group_body, 0)

        pltpu.sync_copy(
            acc_v, out_hbm.at[pl.ds(r0, ROWS_PER_TILE), pl.ds(c0, COLS_PER_PASS)])


_F8 = jnp.float8_e4m3fn
_SCL = {1: 16.0, 2: 128.0, 3: 256.0, 4: 256.0, 8: 256.0, 12: 256.0}


def _norm_kernel(a_ref, p_ref):
    a = a_ref[...]
    deg = jnp.sum(a, axis=1, keepdims=True)
    p_ref[...] = (a * (_SCL[1] / jnp.maximum(deg, 1.0))).astype(_F8)


def _normalize(a):
    return pl.pallas_call(
        _norm_kernel,
        grid=(8,),
        in_specs=[pl.BlockSpec((N // 8, N), lambda i: (i, 0))],
        out_specs=pl.BlockSpec((N // 8, N), lambda i: (i, 0)),
        out_shape=jax.ShapeDtypeStruct((N, N), _F8),
    )(a)


_MB = 1024


def _mm_kernel(rescale, a_ref, b_ref, o_ref, acc_ref):
    @pl.when(pl.program_id(2) == 0)
    def _():
        acc_ref[...] = jnp.zeros_like(acc_ref)

    acc_ref[...] += jnp.dot(a_ref[...], b_ref[...],
                            preferred_element_type=jnp.float32)

    @pl.when(pl.program_id(2) == pl.num_programs(2) - 1)
    def _():
        o_ref[...] = (acc_ref[...] * rescale).astype(_F8)


def _mm(a, b, rescale):
    g = N // _MB
    return pl.pallas_call(
        functools.partial(_mm_kernel, rescale),
        grid=(g, g, g),
        in_specs=[pl.BlockSpec((_MB, _MB), lambda i, j, k: (i, k)),
                  pl.BlockSpec((_MB, _MB), lambda i, j, k: (k, j))],
        out_specs=pl.BlockSpec((_MB, _MB), lambda i, j, k: (i, j)),
        out_shape=jax.ShapeDtypeStruct((N, N), _F8),
        scratch_shapes=[pltpu.VMEM((_MB, _MB), jnp.float32)],
    )(a, b)


def _mm2_kernel(rs1, rs2, a_ref, b1_ref, b2_ref, o1_ref, o2_ref, acc1_ref,
                acc2_ref):
    @pl.when(pl.program_id(2) == 0)
    def _():
        acc1_ref[...] = jnp.zeros_like(acc1_ref)
        acc2_ref[...] = jnp.zeros_like(acc2_ref)

    a = a_ref[...]
    acc1_ref[...] += jnp.dot(a, b1_ref[...],
                             preferred_element_type=jnp.float32)
    acc2_ref[...] += jnp.dot(a, b2_ref[...],
                             preferred_element_type=jnp.float32)

    @pl.when(pl.program_id(2) == pl.num_programs(2) - 1)
    def _():
        o1_ref[...] = (acc1_ref[...] * rs1).astype(_F8)
        o2_ref[...] = (acc2_ref[...] * rs2).astype(_F8)


def _mm2(a, b1, b2, rs1, rs2):
    g = N // _MB
    sd = jax.ShapeDtypeStruct((N, N), _F8)
    return pl.pallas_call(
        functools.partial(_mm2_kernel, rs1, rs2),
        grid=(g, g, g),
        in_specs=[pl.BlockSpec((_MB, _MB), lambda i, j, k: (i, k)),
                  pl.BlockSpec((_MB, _MB), lambda i, j, k: (k, j)),
                  pl.BlockSpec((_MB, _MB), lambda i, j, k: (k, j))],
        out_specs=[pl.BlockSpec((_MB, _MB), lambda i, j, k: (i, j)),
                   pl.BlockSpec((_MB, _MB), lambda i, j, k: (i, j))],
        out_shape=[sd, sd],
        scratch_shapes=[pltpu.VMEM((_MB, _MB), jnp.float32),
                        pltpu.VMEM((_MB, _MB), jnp.float32)],
    )(a, b1, b2)


_BT = 128
_DK = 512


def _diag_kernel(p1_ref, p2_ref, p3_ref, p4r_ref, p4c_ref, p8c_ref, p12c_ref,
                 o_ref, acc_ref):
    bi = pl.program_id(0)
    kk = pl.program_id(1)
    nk = pl.num_programs(1)

    @pl.when(kk == 0)
    def _():
        o_ref[...] = jnp.zeros_like(o_ref)
        acc_ref[...] = jnp.zeros_like(acc_ref)

    a_chunks = [p1_ref[...], p2_ref[...], p3_ref[...], p4r_ref[...]]
    b4 = p4c_ref[...]
    b8 = p8c_ref[...]
    b12 = p12c_ref[...]

    def dot(a, b):
        return jnp.dot(a, b, preferred_element_type=jnp.float32)

    for t, a in enumerate(a_chunks[:3]):
        acc_ref[t * _BT:(t + 1) * _BT, :] += dot(a, b4)
        acc_ref[(3 + t) * _BT:(4 + t) * _BT, :] += dot(a, b8)
    for t, a in enumerate(a_chunks):
        acc_ref[(6 + t) * _BT:(7 + t) * _BT, :] += dot(a, b12)

    ii = lax.broadcasted_iota(jnp.int32, (_BT, _BT), 0)
    jj = lax.broadcasted_iota(jnp.int32, (_BT, _BT), 1)
    eye = (ii == jj).astype(jnp.float32)

    @pl.when(kk == bi // 4)
    def _():
        off = (bi % 4) * _BT
        iw = lax.broadcasted_iota(jnp.int32, (_BT, _DK), 0)
        jw = lax.broadcasted_iota(jnp.int32, (_BT, _DK), 1)
        mask_a = (jw == iw + off).astype(jnp.float32)
        it = lax.broadcasted_iota(jnp.int32, (_DK, _BT), 0)
        jt = lax.broadcasted_iota(jnp.int32, (_DK, _BT), 1)
        mask_b = (it == jt + off).astype(jnp.float32)
        z = jnp.zeros((_BT, 1), jnp.float32)

        def rs_a(x, scl):
            return jnp.sum(x.astype(jnp.float32) * mask_a, axis=1,
                           keepdims=True) * (1.0 / scl)

        def rs_b(x, scl):
            return (jnp.sum(x.astype(jnp.float32) * mask_b, axis=0)[:, None]
                    * (1.0 / scl))

        o_ref[...] += jnp.concatenate(
            [rs_a(a_chunks[0], _SCL[1]), rs_a(a_chunks[1], _SCL[2]),
             rs_a(a_chunks[2], _SCL[3]), rs_a(a_chunks[3], _SCL[4]),
             z, z, z, rs_b(b8, _SCL[8]), z, z, z, rs_b(b12, _SCL[12]),
             z, z, z, z], axis=1)

    @pl.when(kk == nk - 1)
    def _():
        acc = acc_ref[...]
        a_scl = [_SCL[1], _SCL[2], _SCL[3]]

        def dg(t, scl):
            sub = acc[t * _BT:(t + 1) * _BT, :]
            return jnp.sum(sub * eye, axis=1, keepdims=True) * (1.0 / scl)

        z = jnp.zeros((_BT, 1), jnp.float32)
        o_ref[...] += jnp.concatenate(
            [z, z, z, z,
             dg(0, a_scl[0] * _SCL[4]), dg(1, a_scl[1] * _SCL[4]),
             dg(2, a_scl[2] * _SCL[4]), z,
             dg(3, a_scl[0] * _SCL[8]), dg(4, a_scl[1] * _SCL[8]),
             dg(5, a_scl[2] * _SCL[8]), z,
             dg(6, a_scl[0] * _SCL[12]), dg(7, a_scl[1] * _SCL[12]),
             dg(8, a_scl[2] * _SCL[12]), dg(9, _SCL[4] * _SCL[12])],
            axis=1)


def _diag_pairs(p1, p2, p3, p4, p8, p12):
    row = pl.BlockSpec((_BT, _DK), lambda i, k: (i, k))
    col = pl.BlockSpec((_DK, _BT), lambda i, k: (k, i))
    return pl.pallas_call(
        _diag_kernel,
        grid=(N // _BT, N // _DK),
        in_specs=[row, row, row, row, col, col, col],
        out_specs=pl.BlockSpec((_BT, WALK), lambda i, k: (i, 0)),
        out_shape=jax.ShapeDtypeStruct((N, WALK), jnp.float32),
        scratch_shapes=[pltpu.VMEM((10 * _BT, _BT), jnp.float32)],
    )(p1, p2, p3, p4, p4, p8, p12)


_RB = 256


def _cdot(a, b):
    return lax.dot_general(a, b, (((1,), (1,)), ((), ())),
                           preferred_element_type=jnp.float32)


def _inproj_kernel(x_ref, dg_ref, wr_ref, br_ref, w1_ref, w2_ref, b_ref,
                   wc_ref, bc_ref, h_ref, q_ref):
    pe = _cdot(dg_ref[...], wr_ref[...]) + br_ref[...]
    h = _cdot(x_ref[...], w1_ref[...]) + _cdot(pe, w2_ref[...]) + b_ref[...]
    h_ref[...] = h
    q_ref[...] = _cdot(h, wc_ref[...]) + bc_ref[...]


def _input_proj(x, diags, w_rwse, b_rwse, w1, w2, b_in, wcat, bcat):
    return pl.pallas_call(
        _inproj_kernel,
        grid=(N // _RB,),
        in_specs=[
            pl.BlockSpec((_RB, D_FEAT), lambda i: (i, 0)),
            pl.BlockSpec((_RB, WALK), lambda i: (i, 0)),
            pl.BlockSpec((RWSE_DIM, WALK), lambda i: (0, 0)),
            pl.BlockSpec((1, RWSE_DIM), lambda i: (0, 0)),
            pl.BlockSpec((HID, D_FEAT), lambda i: (0, 0)),
            pl.BlockSpec((HID, RWSE_DIM), lambda i: (0, 0)),
            pl.BlockSpec((1, HID), lambda i: (0, 0)),
            pl.BlockSpec((4 * HID, HID), lambda i: (0, 0)),
            pl.BlockSpec((1, 4 * HID), lambda i: (0, 0)),
        ],
        out_specs=[pl.BlockSpec((_RB, HID), lambda i: (i, 0)),
                   pl.BlockSpec((_RB, 4 * HID), lambda i: (i, 0))],
        out_shape=[jax.ShapeDtypeStruct((N, HID), jnp.float32),
                   jax.ShapeDtypeStruct((N, 4 * HID), jnp.float32)],
    )(x, diags, w_rwse, b_rwse, w1, w2, b_in, wcat, bcat)


_BA = 1024


def _attn_accum(q_ref, kv_ref, a_ref, num_ref, den_ref):
    @pl.when(pl.program_id(1) == 0)
    def _():
        num_ref[...] = jnp.zeros_like(num_ref)
        den_ref[...] = jnp.zeros_like(den_ref)

    at = a_ref[...]
    for h in range(HEADS):
        sl = slice(h * DH, (h + 1) * DH)
        ksl = slice(HID + h * DH, HID + (h + 1) * DH)
        vsl = slice(2 * HID + h * DH, 2 * HID + (h + 1) * DH)
        st = _cdot(kv_ref[:, ksl], q_ref[:, sl])
        w = at * jnp.exp(st * 0.25)
        num_ref[:, sl] += lax.dot_general(
            w, kv_ref[:, vsl], (((0,), (0,)), ((), ())),
            preferred_element_type=jnp.float32)
        den_ref[:, h:h + 1] += jnp.sum(w, axis=0)[:, None]


def _conv_core(h_ref, s_ref, num, den, g_ref, b_ref, rows):
    denr = jnp.concatenate(
        [jnp.broadcast_to(den[:, h:h + 1], (rows, DH)) for h in range(HEADS)],
        axis=1)
    conv = num / (denr + 1e-16) + s_ref[:, 3 * HID:4 * HID]
    z = h_ref[...] + conv
    mu = jnp.mean(z, axis=1, keepdims=True)
    zc = z - mu
    var = jnp.mean(zc * zc, axis=1, keepdims=True)
    hn = zc / jnp.sqrt(var + 1e-5) * g_ref[...] + b_ref[...]
    return jnp.maximum(hn, 0.0)


def _attn_proj_kernel(q_ref, kv_ref, a_ref, h_ref, g_ref, b_ref, wc_ref,
                      bc_ref, o_ref, qo_ref, num_ref, den_ref):
    _attn_accum(q_ref, kv_ref, a_ref, num_ref, den_ref)

    @pl.when(pl.program_id(1) == pl.num_programs(1) - 1)
    def _():
        res = _conv_core(h_ref, q_ref, num_ref[...], den_ref[...], g_ref,
                         b_ref, _BA)
        o_ref[...] = res
        qo_ref[...] = _cdot(res, wc_ref[...]) + bc_ref[...]


def _attn_pad_kernel(q_ref, kv_ref, a_ref, h_ref, g_ref, b_ref, o_ref,
                     num_ref, den_ref):
    _attn_accum(q_ref, kv_ref, a_ref, num_ref, den_ref)

    @pl.when(pl.program_id(1) == pl.num_programs(1) - 1)
    def _():
        res = _conv_core(h_ref, q_ref, num_ref[...], den_ref[...], g_ref,
                         b_ref, _BA)
        o_ref[...] = jnp.concatenate([res, jnp.zeros_like(res)], axis=1)


_ATTN_IN = [
    pl.BlockSpec((_BA, 4 * HID), lambda i, j: (i, 0)),
    pl.BlockSpec((_BA, 4 * HID), lambda i, j: (j, 0)),
    pl.BlockSpec((_BA, _BA), lambda i, j: (j, i)),
    pl.BlockSpec((_BA, HID), lambda i, j: (i, 0)),
    pl.BlockSpec((1, HID), lambda i, j: (0, 0)),
    pl.BlockSpec((1, HID), lambda i, j: (0, 0)),
]
_ATTN_SCRATCH = [pltpu.VMEM((_BA, HID), jnp.float32),
                 pltpu.VMEM((_BA, HEADS), jnp.float32)]


def _attn_proj(qkvs, a, h, g, be, wcat, bcat):
    gr = N // _BA
    return pl.pallas_call(
        _attn_proj_kernel,
        grid=(gr, gr),
        in_specs=_ATTN_IN + [
            pl.BlockSpec((4 * HID, HID), lambda i, j: (0, 0)),
            pl.BlockSpec((1, 4 * HID), lambda i, j: (0, 0)),
        ],
        out_specs=[pl.BlockSpec((_BA, HID), lambda i, j: (i, 0)),
                   pl.BlockSpec((_BA, 4 * HID), lambda i, j: (i, 0))],
        out_shape=[jax.ShapeDtypeStruct((N, HID), jnp.float32),
                   jax.ShapeDtypeStruct((N, 4 * HID), jnp.float32)],
        scratch_shapes=_ATTN_SCRATCH,
        compiler_params=pltpu.CompilerParams(
            fuse_transposed_lhs_in_matmul=True),
    )(qkvs, qkvs, a, h, g, be, wcat, bcat)


def _attn_pad(qkvs, a, h, g, be):
    gr = N // _BA
    return pl.pallas_call(
        _attn_pad_kernel,
        grid=(gr, gr),
        in_specs=_ATTN_IN,
        out_specs=pl.BlockSpec((_BA, 2 * HID), lambda i, j: (i, 0)),
        out_shape=jax.ShapeDtypeStruct((N, 2 * HID), jnp.float32),
        scratch_shapes=_ATTN_SCRATCH,
        compiler_params=pltpu.CompilerParams(
            fuse_transposed_lhs_in_matmul=True),
    )(qkvs, qkvs, a, h, g, be)


PAIRS_PER = NPAIRS // 32


@functools.cache
def _pairs_kernel():
    return functools.partial(
        pl.kernel,
        out_type=jax.ShapeDtypeStruct((NPAIRS,), jnp.float32),
        mesh=_sc_mesh(),
        scratch_types=[
            pltpu.VMEM((PAIRS_PER,), jnp.int32),
            pltpu.VMEM((PAIRS_PER,), jnp.int32),
            pltpu.VMEM((PAIRS_PER, 2 * HID), jnp.float32),
            pltpu.VMEM((PAIRS_PER, 2 * HID), jnp.float32),
            pltpu.VMEM((PAIRS_PER,), jnp.float32),
            pltpu.SemaphoreType.DMA,
        ],
        compiler_params=_SC_PARAMS,
    )(_pairs_body)


def _pairs(h, src, dst):
    return _pairs_kernel()(h, src, dst)


def _pairs_body(h_hbm, src_hbm, dst_hbm, out_hbm, s_v, d_v, hs_v, hd_v, res_v,
                sem):
    wid = lax.axis_index("s") * 2 + lax.axis_index("c")
    base = wid * PAIRS_PER
    pltpu.sync_copy(src_hbm.at[pl.ds(base, PAIRS_PER)], s_v)
    pltpu.sync_copy(dst_hbm.at[pl.ds(base, PAIRS_PER)], d_v)
    pltpu.async_copy(h_hbm.at[s_v], hs_v, sem).wait()
    pltpu.async_copy(h_hbm.at[d_v], hd_v, sem).wait()

    def group_body(g, _):
        rows = g * 16 + lax.iota(jnp.int32, 16)
        acc = jnp.zeros((16,), jnp.float32)
        for c in range(HID):
            colv = jnp.full((16,), c, jnp.int32)
            acc = acc + (plsc.load_gather(hs_v, [rows, colv])
                         * plsc.load_gather(hd_v, [rows, colv]))
        res_v[pl.ds(g * 16, 16)] = 1.0 / (1.0 + jnp.exp(-acc))
        return 0

    lax.fori_loop(0, PAIRS_PER // 16, group_body, 0)
    pltpu.sync_copy(res_v, out_hbm.at[pl.ds(base, PAIRS_PER)])


def kernel(x, edge_index, src, dst, W_rwse, b_rwse, W_in, b_in,
           Wq0, bq0, Wk0, bk0, Wv0, bv0, Ws0, bs0, g0, be0,
           Wq1, bq1, Wk1, bk1, Wv1, bv1, Ws1, bs1, g1, be1):
    row = edge_index[0]
    col = edge_index[1]
    a = _build_counts(row, col)
    p = _normalize(a)
    p2 = _mm(p, p, _SCL[2] / (_SCL[1] * _SCL[1]))
    p3, p4 = _mm2(p2, p, p2, _SCL[3] / (_SCL[2] * _SCL[1]),
                  _SCL[4] / (_SCL[2] * _SCL[2]))
    p8 = _mm(p4, p4, _SCL[8] / (_SCL[4] * _SCL[4]))
    p12 = _mm(p8, p4, _SCL[12] / (_SCL[8] * _SCL[4]))
    diags = _diag_pairs(p, p2, p3, p4, p8, p12)
    wcat0 = jnp.concatenate([Wq0, Wk0, Wv0, Ws0], axis=0)
    bcat0 = jnp.concatenate([bq0, bk0, bv0, bs0]).reshape(1, -1)
    wcat1 = jnp.concatenate([Wq1, Wk1, Wv1, Ws1], axis=0)
    bcat1 = jnp.concatenate([bq1, bk1, bv1, bs1]).reshape(1, -1)
    h, qkvs0 = _input_proj(x, diags, W_rwse, b_rwse.reshape(1, -1),
                           W_in[:, :D_FEAT], W_in[:, D_FEAT:],
                           b_in.reshape(1, -1), wcat0, bcat0)
    h, qkvs1 = _attn_proj(qkvs0, a, h, g0.reshape(1, -1),
                          be0.reshape(1, -1), wcat1, bcat1)
    hp = _attn_pad(qkvs1, a, h, g1.reshape(1, -1), be1.reshape(1, -1))
    return _pairs(hp, src, dst)
